# baseline (speedup 1.0000x reference)
import jax
import jax.numpy as jnp
from jax import lax
from jax.experimental import pallas as pl
from jax.experimental.pallas import tpu as pltpu
from jax.experimental.pallas import tpu_sc as plsc

N = 10000
NPAD = 10240
E = 320000
D_IN = 128
D_H = 256
D_OUT = 128
BATCH = 8192
IDS = 3 * BATCH

NC, NS = 2, 16
NW = NC * NS
C = 80
NCH = 128
EPW = NCH * C
EPAD = NW * EPW
RPT = NPAD // NS
RC = 80
IPW = IDS // NW
GC = 128

_mesh = plsc.VectorSubcoreMesh(
    core_axis_name="c", subcore_axis_name="s", num_cores=NC, num_subcores=NS)


def _make_edge_agg():
  scratch = [
      pltpu.VMEM((C,), jnp.int32),
      pltpu.VMEM((C,), jnp.int32),
      pltpu.VMEM((C,), jnp.int32),
      pltpu.VMEM((C,), jnp.int32),
      pltpu.VMEM((C, 128), jnp.float32),
      pltpu.VMEM((C, 128), jnp.float32),
      pltpu.VMEM((RC, 128), jnp.float32),
      pltpu.VMEM_SHARED((NPAD, 128), jnp.float32),
      pltpu.SemaphoreType.DMA,
      pltpu.SemaphoreType.DMA,
      pltpu.SemaphoreType.DMA,
      pltpu.SemaphoreType.DMA,
  ]

  def body(x_hbm, src_hbm, dst_hbm, agg_out, src_v0, src_v1, dst_v0, dst_v1,
           rows0, rows1, zbuf, acc_sh, sg0, sg1, ss0, ss1):
    src_v = (src_v0, src_v1)
    dst_v = (dst_v0, dst_v1)
    rows = (rows0, rows1)
    sg = (sg0, sg1)
    ss = (ss0, ss1)
    cid = lax.axis_index("c")
    sid = lax.axis_index("s")
    wid = sid * NC + cid
    z16 = jnp.zeros((16,), jnp.float32)

    def zrow(i, carry):
      for k in range(8):
        zbuf[i, pl.ds(k * 16, 16)] = z16
      return carry
    lax.fori_loop(0, RC, zrow, 0)
    row0 = sid * RPT
    for j in range(RPT // RC):
      pltpu.sync_copy(zbuf, acc_sh.at[pl.ds(row0 + j * RC, RC)])
    plsc.subcore_barrier()

    ebase = wid * EPW

    def ld_idx(i, b):
      pltpu.sync_copy(src_hbm.at[pl.ds(ebase + i * C, C)], src_v[b])
      pltpu.sync_copy(dst_hbm.at[pl.ds(ebase + i * C, C)], dst_v[b])

    def g_start(b):
      pltpu.async_copy(x_hbm.at[src_v[b]], rows[b], sg[b])

    def g_wait(b):
      pltpu.make_async_copy(x_hbm.at[src_v[b]], rows[b], sg[b]).wait()

    def s_start(b):
      pltpu.async_copy(rows[b], acc_sh.at[dst_v[b]], ss[b], add=True)

    def s_wait(b):
      pltpu.make_async_copy(rows[b], acc_sh.at[dst_v[b]], ss[b]).wait()

    T = NCH // 2
    ld_idx(0, 0)
    g_start(0)

    def super_chunk(t, carry):
      i0 = 2 * t

      @pl.when(t > 0)
      def _():
        s_wait(1)
      ld_idx(i0 + 1, 1)
      g_start(1)
      g_wait(0)
      s_start(0)
      g_wait(1)
      s_start(1)
      s_wait(0)

      @pl.when(t + 1 < T)
      def _():
        ld_idx(i0 + 2, 0)
        g_start(0)
      return carry
    lax.fori_loop(0, T, super_chunk, 0)
    s_wait(1)
    plsc.subcore_barrier()

    obase = cid * NPAD + row0
    for j in range(RPT // RC):
      pltpu.sync_copy(acc_sh.at[pl.ds(row0 + j * RC, RC)], zbuf)
      pltpu.sync_copy(zbuf, agg_out.at[pl.ds(obase + j * RC, RC)])

  return pl.kernel(
      body,
      out_type=jax.ShapeDtypeStruct((NC * NPAD, 128), jnp.float32),
      mesh=_mesh,
      scratch_types=scratch,
      compiler_params=pltpu.CompilerParams(use_tc_tiling_on_sc=False),
  )


def _deg_body(dst_hbm, deg_out, dst_all, ones_v, degst, deg_sh, sem):
  cid = lax.axis_index("c")
  sid = lax.axis_index("s")
  wid = sid * NC + cid
  z16 = jnp.zeros((16,), jnp.float32)
  row0 = sid * RPT

  pltpu.sync_copy(dst_hbm.at[wid], dst_all)

  def onesrow(i, carry):
    ones_v[i] = jnp.full((16,), 1.0, jnp.float32)
    return carry
  lax.fori_loop(0, C, onesrow, 0)

  def zdrow(i, carry):
    degst[i] = z16
    return carry
  lax.fori_loop(0, RPT, zdrow, 0)
  pltpu.sync_copy(degst, deg_sh.at[pl.ds(row0, RPT)])
  plsc.subcore_barrier()

  def chunk(i, carry):
    pltpu.async_copy(ones_v, deg_sh.at[dst_all.at[i]], sem, add=True)
    pltpu.make_async_copy(ones_v, deg_sh.at[dst_all.at[i]], sem).wait()
    return carry
  lax.fori_loop(0, NCH, chunk, 0)
  plsc.subcore_barrier()

  obase = cid * NPAD + row0
  pltpu.sync_copy(deg_sh.at[pl.ds(row0, RPT)], degst)
  pltpu.sync_copy(degst, deg_out.at[pl.ds(obase, RPT)])


_deg_count = pl.kernel(
    _deg_body,
    out_type=jax.ShapeDtypeStruct((NC * NPAD, 16), jnp.float32),
    mesh=_mesh,
    scratch_types=[
        pltpu.VMEM((NCH, C), jnp.int32),
        pltpu.VMEM((C, 16), jnp.float32),
        pltpu.VMEM((RPT, 16), jnp.float32),
        pltpu.VMEM_SHARED((NPAD, 16), jnp.float32),
        pltpu.SemaphoreType.DMA,
    ],
    compiler_params=pltpu.CompilerParams(use_tc_tiling_on_sc=False),
)

_edge_agg = _make_edge_agg()


def _gather_body(h_hbm, ids_hbm, out_hbm, idx_v, rows_v, sem):
  wid = lax.axis_index("s") * NC + lax.axis_index("c")
  base = wid * IPW
  for j in range(IPW // GC):
    pltpu.sync_copy(ids_hbm.at[pl.ds(base + j * GC, GC)], idx_v)
    pltpu.async_copy(h_hbm.at[idx_v], rows_v, sem).wait()
    pltpu.sync_copy(rows_v, out_hbm.at[pl.ds(base + j * GC, GC)])


_gather_rows = pl.kernel(
    _gather_body,
    out_type=jax.ShapeDtypeStruct((IDS, 128), jnp.float32),
    mesh=_mesh,
    scratch_types=[
        pltpu.VMEM((GC,), jnp.int32),
        pltpu.VMEM((GC, 128), jnp.float32),
        pltpu.SemaphoreType.DMA,
    ],
    compiler_params=pltpu.CompilerParams(use_tc_tiling_on_sc=False),
)


RB = 1280


def _layer_kernel(aggp, degp, nf, wn0, wr0, b0, wn1, wr1, b1, y_ref, xr_ref):
  agg = aggp[0] + aggp[1]
  deg = degp[0, :, 0:1] + degp[1, :, 0:1]
  rd = 1.0 / jnp.maximum(deg, 1.0)
  mean0 = agg * rd
  h = jnp.dot(mean0, wn0[...], preferred_element_type=jnp.float32)
  h = h + jnp.dot(nf[...], wr0[...], preferred_element_type=jnp.float32)
  h = jnp.maximum(h + b0[...], 0.0)
  y_ref[...] = jnp.dot(h, wn1[...], preferred_element_type=jnp.float32)
  xr_ref[...] = jnp.dot(h, wr1[...], preferred_element_type=jnp.float32) + b1[...]


def _layer_call(aggp, degp, nf, wn0, wr0, b0, wn1, wr1, b1):
  return pl.pallas_call(
      _layer_kernel,
      grid=(NPAD // RB,),
      in_specs=[
          pl.BlockSpec((NC, RB, 128), lambda i: (0, i, 0)),
          pl.BlockSpec((NC, RB, 16), lambda i: (0, i, 0)),
          pl.BlockSpec((RB, 128), lambda i: (i, 0)),
          pl.BlockSpec((D_IN, D_H), lambda i: (0, 0)),
          pl.BlockSpec((D_IN, D_H), lambda i: (0, 0)),
          pl.BlockSpec((1, D_H), lambda i: (0, 0)),
          pl.BlockSpec((D_H, D_OUT), lambda i: (0, 0)),
          pl.BlockSpec((D_H, D_OUT), lambda i: (0, 0)),
          pl.BlockSpec((1, D_OUT), lambda i: (0, 0)),
      ],
      out_specs=[
          pl.BlockSpec((RB, 128), lambda i: (i, 0)),
          pl.BlockSpec((RB, 128), lambda i: (i, 0)),
      ],
      out_shape=[
          jax.ShapeDtypeStruct((NPAD, 128), jnp.float32),
          jax.ShapeDtypeStruct((NPAD, 128), jnp.float32),
      ],
  )(aggp, degp, nf, wn0, wr0, b0, wn1, wr1, b1)


def _h1_kernel(aggp, degp, xr, out_ref):
  agg = aggp[0] + aggp[1]
  deg = degp[0, :, 0:1] + degp[1, :, 0:1]
  out_ref[...] = agg * (1.0 / jnp.maximum(deg, 1.0)) + xr[...]


def _h1_call(aggp, degp, xr):
  return pl.pallas_call(
      _h1_kernel,
      grid=(NPAD // RB,),
      in_specs=[
          pl.BlockSpec((NC, RB, 128), lambda i: (0, i, 0)),
          pl.BlockSpec((NC, RB, 16), lambda i: (0, i, 0)),
          pl.BlockSpec((RB, 128), lambda i: (i, 0)),
      ],
      out_specs=pl.BlockSpec((RB, 128), lambda i: (i, 0)),
      out_shape=jax.ShapeDtypeStruct((NPAD, 128), jnp.float32),
  )(aggp, degp, xr)


def _score_kernel(f, wp, pos_ref, neg_ref):
  s = f[0:BATCH]
  p = f[BATCH:2 * BATCH]
  n = f[2 * BATCH:3 * BATCH]
  w = wp[...]
  pos_ref[...] = jnp.sum(s * p * w, axis=1, keepdims=True)
  neg_ref[...] = jnp.sum(s * n * w, axis=1, keepdims=True)


def _score_call(feats, wp):
  return pl.pallas_call(
      _score_kernel,
      out_shape=[
          jax.ShapeDtypeStruct((BATCH, 1), jnp.float32),
          jax.ShapeDtypeStruct((BATCH, 1), jnp.float32),
      ],
  )(feats, wp)


def kernel(src_ids, pos_dst_ids, neg_dst_ids, node_feat, edge_index,
           Wn0, Wr0, b0, Wn1, Wr1, b1, w_pred):
  f32 = jnp.float32
  nf_pad = jnp.concatenate(
      [node_feat.astype(f32), jnp.zeros((NPAD - N, D_IN), f32)], axis=0)
  pad = jnp.full((EPAD - E,), NPAD - 1, jnp.int32)
  src = jnp.concatenate([edge_index[0].astype(jnp.int32), pad])
  dst = jnp.concatenate([edge_index[1].astype(jnp.int32), pad])
  dst3 = dst.reshape(NW, NCH, C)
  ids = jnp.concatenate([src_ids, pos_dst_ids, neg_dst_ids]).astype(jnp.int32)

  degpf = _deg_count(dst3)
  src, dst, degpf = lax.optimization_barrier((src, dst, degpf))
  degp = degpf.reshape(NC, NPAD, 16)
  aggp0 = _edge_agg(nf_pad, src, dst).reshape(NC, NPAD, 128)

  y, xr = _layer_call(aggp0, degp, nf_pad, Wn0, Wr0,
                      b0.reshape(1, -1), Wn1, Wr1, b1.reshape(1, -1))

  aggp1 = _edge_agg(y, src, dst).reshape(NC, NPAD, 128)
  h1 = _h1_call(aggp1, degp, xr)

  feats = _gather_rows(h1, ids)
  pos, neg = _score_call(feats, w_pred.reshape(1, -1))
  return (pos.reshape(-1), neg.reshape(-1))


# speedup vs baseline: 2.1564x; 2.1564x over previous
import jax
import jax.numpy as jnp
from jax import lax
from jax.experimental import pallas as pl
from jax.experimental.pallas import tpu as pltpu
from jax.experimental.pallas import tpu_sc as plsc

N = 10000
NPAD = 10240
E = 320000
D_IN = 128
D_H = 256
D_OUT = 128
BATCH = 8192
IDS = 3 * BATCH

NC, NS = 2, 16
NW = NC * NS
C = 80
NCH = 128
EPW = NCH * C
EPAD = NW * EPW
RPT = NPAD // NS
RC = 80
IPW = IDS // NW
GC = 128

_mesh = plsc.VectorSubcoreMesh(
    core_axis_name="c", subcore_axis_name="s", num_cores=NC, num_subcores=NS)


def _make_edge_agg():
  scratch = [
      pltpu.VMEM((C,), jnp.int32),
      pltpu.VMEM((C,), jnp.int32),
      pltpu.VMEM((C,), jnp.int32),
      pltpu.VMEM((C,), jnp.int32),
      pltpu.VMEM((C, 128), jnp.float32),
      pltpu.VMEM((C, 128), jnp.float32),
      pltpu.VMEM((RC, 128), jnp.float32),
      pltpu.VMEM_SHARED((NPAD, 128), jnp.float32),
      pltpu.SemaphoreType.DMA,
      pltpu.SemaphoreType.DMA,
      pltpu.SemaphoreType.DMA,
      pltpu.SemaphoreType.DMA,
  ]

  def body(x_hbm, src_hbm, dst_hbm, agg_out, src_v0, src_v1, dst_v0, dst_v1,
           rows0, rows1, zbuf, acc_sh, sg0, sg1, ss0, ss1):
    src_v = (src_v0, src_v1)
    dst_v = (dst_v0, dst_v1)
    rows = (rows0, rows1)
    sg = (sg0, sg1)
    ss = (ss0, ss1)
    cid = lax.axis_index("c")
    sid = lax.axis_index("s")
    wid = sid * NC + cid
    z16 = jnp.zeros((16,), jnp.float32)

    def zrow(i, carry):
      for k in range(8):
        zbuf[i, pl.ds(k * 16, 16)] = z16
      return carry
    lax.fori_loop(0, RC, zrow, 0)
    row0 = sid * RPT
    for j in range(RPT // RC):
      pltpu.sync_copy(zbuf, acc_sh.at[pl.ds(row0 + j * RC, RC)])
    plsc.subcore_barrier()

    ebase = wid * EPW

    def ld_idx(i, b):
      pltpu.sync_copy(src_hbm.at[pl.ds(ebase + i * C, C)], src_v[b])
      pltpu.sync_copy(dst_hbm.at[pl.ds(ebase + i * C, C)], dst_v[b])

    def g_start(b):
      pltpu.async_copy(x_hbm.at[src_v[b]], rows[b], sg[b])

    def g_wait(b):
      pltpu.make_async_copy(x_hbm.at[src_v[b]], rows[b], sg[b]).wait()

    def s_start(b):
      pltpu.async_copy(rows[b], acc_sh.at[dst_v[b]], ss[b], add=True)

    def s_wait(b):
      pltpu.make_async_copy(rows[b], acc_sh.at[dst_v[b]], ss[b]).wait()

    T = NCH // 2
    ld_idx(0, 0)
    g_start(0)

    def super_chunk(t, carry):
      i0 = 2 * t

      @pl.when(t > 0)
      def _():
        s_wait(1)
      ld_idx(i0 + 1, 1)
      g_start(1)
      g_wait(0)
      s_start(0)
      g_wait(1)
      s_start(1)
      s_wait(0)

      @pl.when(t + 1 < T)
      def _():
        ld_idx(i0 + 2, 0)
        g_start(0)
      return carry
    lax.fori_loop(0, T, super_chunk, 0)
    s_wait(1)
    plsc.subcore_barrier()

    obase = cid * NPAD + row0
    for j in range(RPT // RC):
      pltpu.sync_copy(acc_sh.at[pl.ds(row0 + j * RC, RC)], zbuf)
      pltpu.sync_copy(zbuf, agg_out.at[pl.ds(obase + j * RC, RC)])

  return pl.kernel(
      body,
      out_type=jax.ShapeDtypeStruct((NC * NPAD, 128), jnp.float32),
      mesh=_mesh,
      scratch_types=scratch,
      compiler_params=pltpu.CompilerParams(use_tc_tiling_on_sc=False),
  )


def _deg_body(dst_hbm, deg_out, dst_all, ones_v, degst, deg_sh, sem):
  cid = lax.axis_index("c")
  sid = lax.axis_index("s")
  wid = sid * NC + cid
  z16 = jnp.zeros((16,), jnp.float32)
  row0 = sid * RPT

  pltpu.sync_copy(dst_hbm.at[wid], dst_all)

  def onesrow(i, carry):
    ones_v[i] = jnp.full((16,), 1.0, jnp.float32)
    return carry
  lax.fori_loop(0, C, onesrow, 0)

  def zdrow(i, carry):
    degst[i] = z16
    return carry
  lax.fori_loop(0, RPT, zdrow, 0)
  pltpu.sync_copy(degst, deg_sh.at[pl.ds(row0, RPT)])
  plsc.subcore_barrier()

  def chunk(i, carry):
    pltpu.async_copy(ones_v, deg_sh.at[dst_all.at[i]], sem, add=True)
    pltpu.make_async_copy(ones_v, deg_sh.at[dst_all.at[i]], sem).wait()
    return carry
  lax.fori_loop(0, NCH, chunk, 0)
  plsc.subcore_barrier()

  obase = cid * NPAD + row0
  pltpu.sync_copy(deg_sh.at[pl.ds(row0, RPT)], degst)
  pltpu.sync_copy(degst, deg_out.at[pl.ds(obase, RPT)])


_deg_count = pl.kernel(
    _deg_body,
    out_type=jax.ShapeDtypeStruct((NC * NPAD, 16), jnp.float32),
    mesh=_mesh,
    scratch_types=[
        pltpu.VMEM((NCH, C), jnp.int32),
        pltpu.VMEM((C, 16), jnp.float32),
        pltpu.VMEM((RPT, 16), jnp.float32),
        pltpu.VMEM_SHARED((NPAD, 16), jnp.float32),
        pltpu.SemaphoreType.DMA,
    ],
    compiler_params=pltpu.CompilerParams(use_tc_tiling_on_sc=False),
)

_edge_agg = _make_edge_agg()


def _gather_body(h_hbm, ids_hbm, out_hbm, idx_v, rows_v, sem):
  wid = lax.axis_index("s") * NC + lax.axis_index("c")
  base = wid * IPW
  for j in range(IPW // GC):
    pltpu.sync_copy(ids_hbm.at[pl.ds(base + j * GC, GC)], idx_v)
    pltpu.async_copy(h_hbm.at[idx_v], rows_v, sem).wait()
    pltpu.sync_copy(rows_v, out_hbm.at[pl.ds(base + j * GC, GC)])


_gather_rows = pl.kernel(
    _gather_body,
    out_type=jax.ShapeDtypeStruct((IDS, 128), jnp.float32),
    mesh=_mesh,
    scratch_types=[
        pltpu.VMEM((GC,), jnp.int32),
        pltpu.VMEM((GC, 128), jnp.float32),
        pltpu.SemaphoreType.DMA,
    ],
    compiler_params=pltpu.CompilerParams(use_tc_tiling_on_sc=False),
)


RB = 1280


def _layer_kernel(aggp, degp, nf, wn0, wr0, b0, wn1, wr1, b1, y_ref, xr_ref):
  agg = aggp[0] + aggp[1]
  deg = degp[0, :, 0:1] + degp[1, :, 0:1]
  rd = 1.0 / jnp.maximum(deg, 1.0)
  mean0 = agg * rd
  h = jnp.dot(mean0, wn0[...], preferred_element_type=jnp.float32)
  h = h + jnp.dot(nf[...], wr0[...], preferred_element_type=jnp.float32)
  h = jnp.maximum(h + b0[...], 0.0)
  y_ref[...] = jnp.dot(h, wn1[...], preferred_element_type=jnp.float32)
  xr_ref[...] = jnp.dot(h, wr1[...], preferred_element_type=jnp.float32) + b1[...]


def _layer_call(aggp, degp, nf, wn0, wr0, b0, wn1, wr1, b1):
  return pl.pallas_call(
      _layer_kernel,
      grid=(NPAD // RB,),
      in_specs=[
          pl.BlockSpec((NC, RB, 128), lambda i: (0, i, 0)),
          pl.BlockSpec((NC, RB, 16), lambda i: (0, i, 0)),
          pl.BlockSpec((RB, 128), lambda i: (i, 0)),
          pl.BlockSpec((D_IN, D_H), lambda i: (0, 0)),
          pl.BlockSpec((D_IN, D_H), lambda i: (0, 0)),
          pl.BlockSpec((1, D_H), lambda i: (0, 0)),
          pl.BlockSpec((D_H, D_OUT), lambda i: (0, 0)),
          pl.BlockSpec((D_H, D_OUT), lambda i: (0, 0)),
          pl.BlockSpec((1, D_OUT), lambda i: (0, 0)),
      ],
      out_specs=[
          pl.BlockSpec((RB, 128), lambda i: (i, 0)),
          pl.BlockSpec((RB, 128), lambda i: (i, 0)),
      ],
      out_shape=[
          jax.ShapeDtypeStruct((NPAD, 128), jnp.float32),
          jax.ShapeDtypeStruct((NPAD, 128), jnp.float32),
      ],
  )(aggp, degp, nf, wn0, wr0, b0, wn1, wr1, b1)


def _h1_kernel(aggp, degp, xr, out_ref):
  agg = aggp[0] + aggp[1]
  deg = degp[0, :, 0:1] + degp[1, :, 0:1]
  out_ref[...] = agg * (1.0 / jnp.maximum(deg, 1.0)) + xr[...]


def _h1_call(aggp, degp, xr):
  return pl.pallas_call(
      _h1_kernel,
      grid=(NPAD // RB,),
      in_specs=[
          pl.BlockSpec((NC, RB, 128), lambda i: (0, i, 0)),
          pl.BlockSpec((NC, RB, 16), lambda i: (0, i, 0)),
          pl.BlockSpec((RB, 128), lambda i: (i, 0)),
      ],
      out_specs=pl.BlockSpec((RB, 128), lambda i: (i, 0)),
      out_shape=jax.ShapeDtypeStruct((NPAD, 128), jnp.float32),
  )(aggp, degp, xr)


def _score_kernel(f, wp, pos_ref, neg_ref):
  s = f[0:BATCH]
  p = f[BATCH:2 * BATCH]
  n = f[2 * BATCH:3 * BATCH]
  w = wp[...]
  pos_ref[...] = jnp.sum(s * p * w, axis=1, keepdims=True)
  neg_ref[...] = jnp.sum(s * n * w, axis=1, keepdims=True)


def _score_call(feats, wp):
  return pl.pallas_call(
      _score_kernel,
      out_shape=[
          jax.ShapeDtypeStruct((BATCH, 1), jnp.float32),
          jax.ShapeDtypeStruct((BATCH, 1), jnp.float32),
      ],
  )(feats, wp)


def kernel(src_ids, pos_dst_ids, neg_dst_ids, node_feat, edge_index,
           Wn0, Wr0, b0, Wn1, Wr1, b1, w_pred):
  f32 = jnp.float32
  nf_pad = jnp.concatenate(
      [node_feat.astype(f32), jnp.zeros((NPAD - N, D_IN), f32)], axis=0)
  pad = N + (jnp.arange(EPAD - E, dtype=jnp.int32) % (NPAD - N))
  src = jnp.concatenate([edge_index[0].astype(jnp.int32), pad])
  dst = jnp.concatenate([edge_index[1].astype(jnp.int32), pad])
  dst3 = dst.reshape(NW, NCH, C)
  ids = jnp.concatenate([src_ids, pos_dst_ids, neg_dst_ids]).astype(jnp.int32)

  degpf = _deg_count(dst3)
  src, dst, degpf = lax.optimization_barrier((src, dst, degpf))
  degp = degpf.reshape(NC, NPAD, 16)
  aggp0 = _edge_agg(nf_pad, src, dst).reshape(NC, NPAD, 128)

  y, xr = _layer_call(aggp0, degp, nf_pad, Wn0, Wr0,
                      b0.reshape(1, -1), Wn1, Wr1, b1.reshape(1, -1))

  aggp1 = _edge_agg(y, src, dst).reshape(NC, NPAD, 128)
  h1 = _h1_call(aggp1, degp, xr)

  feats = _gather_rows(h1, ids)
  pos, neg = _score_call(feats, w_pred.reshape(1, -1))
  return (pos.reshape(-1), neg.reshape(-1))


# speedup vs baseline: 2.1770x; 1.0096x over previous
import jax
import jax.numpy as jnp
from jax import lax
from jax.experimental import pallas as pl
from jax.experimental.pallas import tpu as pltpu
from jax.experimental.pallas import tpu_sc as plsc

N = 10000
NPAD = 10240
E = 320000
D_IN = 128
D_H = 256
D_OUT = 128
BATCH = 8192
IDS = 3 * BATCH

NC, NS = 2, 16
NW = NC * NS
C = 80
NCH = 128
EPW = NCH * C
EPAD = NW * EPW
RPT = NPAD // NS
RC = 80
IPW = IDS // NW
GC = 128

_mesh = plsc.VectorSubcoreMesh(
    core_axis_name="c", subcore_axis_name="s", num_cores=NC, num_subcores=NS)


def _make_edge_agg():
  scratch = [
      pltpu.VMEM((C,), jnp.int32),
      pltpu.VMEM((C,), jnp.int32),
      pltpu.VMEM((C,), jnp.int32),
      pltpu.VMEM((C,), jnp.int32),
      pltpu.VMEM((C, 128), jnp.float32),
      pltpu.VMEM((C, 128), jnp.float32),
      pltpu.VMEM((RC, 128), jnp.float32),
      pltpu.VMEM_SHARED((NPAD, 128), jnp.float32),
      pltpu.SemaphoreType.DMA,
      pltpu.SemaphoreType.DMA,
      pltpu.SemaphoreType.DMA,
      pltpu.SemaphoreType.DMA,
  ]

  def body(x_hbm, src_hbm, dst_hbm, agg_out, src_v0, src_v1, dst_v0, dst_v1,
           rows0, rows1, zbuf, acc_sh, sg0, sg1, ss0, ss1):
    src_v = (src_v0, src_v1)
    dst_v = (dst_v0, dst_v1)
    rows = (rows0, rows1)
    sg = (sg0, sg1)
    ss = (ss0, ss1)
    cid = lax.axis_index("c")
    sid = lax.axis_index("s")
    wid = sid * NC + cid
    z16 = jnp.zeros((16,), jnp.float32)

    def zrow(i, carry):
      for k in range(8):
        zbuf[i, pl.ds(k * 16, 16)] = z16
      return carry
    lax.fori_loop(0, RC, zrow, 0)
    row0 = sid * RPT
    for j in range(RPT // RC):
      pltpu.sync_copy(zbuf, acc_sh.at[pl.ds(row0 + j * RC, RC)])
    plsc.subcore_barrier()

    ebase = wid * EPW

    def ld_idx(i, b):
      pltpu.sync_copy(src_hbm.at[pl.ds(ebase + i * C, C)], src_v[b])
      pltpu.sync_copy(dst_hbm.at[pl.ds(ebase + i * C, C)], dst_v[b])

    def g_start(b):
      pltpu.async_copy(x_hbm.at[src_v[b]], rows[b], sg[b])

    def g_wait(b):
      pltpu.make_async_copy(x_hbm.at[src_v[b]], rows[b], sg[b]).wait()

    def s_start(b):
      pltpu.async_copy(rows[b], acc_sh.at[dst_v[b]], ss[b], add=True)

    def s_wait(b):
      pltpu.make_async_copy(rows[b], acc_sh.at[dst_v[b]], ss[b]).wait()

    T = NCH // 2
    ld_idx(0, 0)
    g_start(0)

    def super_chunk(t, carry):
      i0 = 2 * t

      @pl.when(t > 0)
      def _():
        s_wait(1)
      ld_idx(i0 + 1, 1)
      g_start(1)
      g_wait(0)
      s_start(0)
      g_wait(1)
      s_start(1)
      s_wait(0)

      @pl.when(t + 1 < T)
      def _():
        ld_idx(i0 + 2, 0)
        g_start(0)
      return carry
    lax.fori_loop(0, T, super_chunk, 0)
    s_wait(1)
    plsc.subcore_barrier()

    obase = cid * NPAD + row0
    for j in range(RPT // RC):
      pltpu.sync_copy(acc_sh.at[pl.ds(row0 + j * RC, RC)], zbuf)
      pltpu.sync_copy(zbuf, agg_out.at[pl.ds(obase + j * RC, RC)])

  return pl.kernel(
      body,
      out_type=jax.ShapeDtypeStruct((NC * NPAD, 128), jnp.float32),
      mesh=_mesh,
      scratch_types=scratch,
      compiler_params=pltpu.CompilerParams(use_tc_tiling_on_sc=False),
  )


def _deg_body(dst_hbm, deg_out, dst_all, ones_v, degst, deg_sh, sem):
  cid = lax.axis_index("c")
  sid = lax.axis_index("s")
  wid = sid * NC + cid
  z16 = jnp.zeros((16,), jnp.float32)
  row0 = sid * RPT

  pltpu.sync_copy(dst_hbm.at[wid], dst_all)

  def onesrow(i, carry):
    ones_v[i] = jnp.full((16,), 1.0, jnp.float32)
    return carry
  lax.fori_loop(0, C, onesrow, 0)

  def zdrow(i, carry):
    degst[i] = z16
    return carry
  lax.fori_loop(0, RPT, zdrow, 0)
  pltpu.sync_copy(degst, deg_sh.at[pl.ds(row0, RPT)])
  plsc.subcore_barrier()

  def chunk(i, carry):
    pltpu.async_copy(ones_v, deg_sh.at[dst_all.at[i]], sem, add=True)
    pltpu.make_async_copy(ones_v, deg_sh.at[dst_all.at[i]], sem).wait()
    return carry
  lax.fori_loop(0, NCH, chunk, 0)
  plsc.subcore_barrier()

  obase = cid * NPAD + row0
  pltpu.sync_copy(deg_sh.at[pl.ds(row0, RPT)], degst)
  pltpu.sync_copy(degst, deg_out.at[pl.ds(obase, RPT)])


_deg_count = pl.kernel(
    _deg_body,
    out_type=jax.ShapeDtypeStruct((NC * NPAD, 16), jnp.float32),
    mesh=_mesh,
    scratch_types=[
        pltpu.VMEM((NCH, C), jnp.int32),
        pltpu.VMEM((C, 16), jnp.float32),
        pltpu.VMEM((RPT, 16), jnp.float32),
        pltpu.VMEM_SHARED((NPAD, 16), jnp.float32),
        pltpu.SemaphoreType.DMA,
    ],
    compiler_params=pltpu.CompilerParams(use_tc_tiling_on_sc=False),
)

_edge_agg = _make_edge_agg()


def _gather_body(h_hbm, ids_hbm, out_hbm, idx_v, rows_v, sem):
  wid = lax.axis_index("s") * NC + lax.axis_index("c")
  base = wid * IPW
  for j in range(IPW // GC):
    pltpu.sync_copy(ids_hbm.at[pl.ds(base + j * GC, GC)], idx_v)
    pltpu.async_copy(h_hbm.at[idx_v], rows_v, sem).wait()
    pltpu.sync_copy(rows_v, out_hbm.at[pl.ds(base + j * GC, GC)])


_gather_rows = pl.kernel(
    _gather_body,
    out_type=jax.ShapeDtypeStruct((IDS, 128), jnp.float32),
    mesh=_mesh,
    scratch_types=[
        pltpu.VMEM((GC,), jnp.int32),
        pltpu.VMEM((GC, 128), jnp.float32),
        pltpu.SemaphoreType.DMA,
    ],
    compiler_params=pltpu.CompilerParams(use_tc_tiling_on_sc=False),
)


RB = 1280


def _layer_kernel(aggp, degp, nf, wn0, wr0, b0, wn1, wr1, b1, y_ref, xr_ref):
  agg = aggp[0] + aggp[1]
  deg = degp[0, :, 0:1] + degp[1, :, 0:1]
  rd = 1.0 / jnp.maximum(deg, 1.0)
  mean0 = agg * rd
  h = jnp.dot(mean0, wn0[...], preferred_element_type=jnp.float32)
  h = h + jnp.dot(nf[...], wr0[...], preferred_element_type=jnp.float32)
  h = jnp.maximum(h + b0[...], 0.0)
  y_ref[...] = jnp.dot(h, wn1[...], preferred_element_type=jnp.float32)
  xr_ref[...] = jnp.dot(h, wr1[...], preferred_element_type=jnp.float32) + b1[...]


def _layer_call(aggp, degp, nf, wn0, wr0, b0, wn1, wr1, b1):
  return pl.pallas_call(
      _layer_kernel,
      grid=(NPAD // RB,),
      in_specs=[
          pl.BlockSpec((NC, RB, 128), lambda i: (0, i, 0)),
          pl.BlockSpec((NC, RB, 16), lambda i: (0, i, 0)),
          pl.BlockSpec((RB, 128), lambda i: (i, 0)),
          pl.BlockSpec((D_IN, D_H), lambda i: (0, 0)),
          pl.BlockSpec((D_IN, D_H), lambda i: (0, 0)),
          pl.BlockSpec((1, D_H), lambda i: (0, 0)),
          pl.BlockSpec((D_H, D_OUT), lambda i: (0, 0)),
          pl.BlockSpec((D_H, D_OUT), lambda i: (0, 0)),
          pl.BlockSpec((1, D_OUT), lambda i: (0, 0)),
      ],
      out_specs=[
          pl.BlockSpec((RB, 128), lambda i: (i, 0)),
          pl.BlockSpec((RB, 128), lambda i: (i, 0)),
      ],
      out_shape=[
          jax.ShapeDtypeStruct((NPAD, 128), jnp.float32),
          jax.ShapeDtypeStruct((NPAD, 128), jnp.float32),
      ],
  )(aggp, degp, nf, wn0, wr0, b0, wn1, wr1, b1)


def _h1_kernel(aggp, degp, xr, out_ref):
  agg = aggp[0] + aggp[1]
  deg = degp[0, :, 0:1] + degp[1, :, 0:1]
  out_ref[...] = agg * (1.0 / jnp.maximum(deg, 1.0)) + xr[...]


def _h1_call(aggp, degp, xr):
  return pl.pallas_call(
      _h1_kernel,
      grid=(NPAD // RB,),
      in_specs=[
          pl.BlockSpec((NC, RB, 128), lambda i: (0, i, 0)),
          pl.BlockSpec((NC, RB, 16), lambda i: (0, i, 0)),
          pl.BlockSpec((RB, 128), lambda i: (i, 0)),
      ],
      out_specs=pl.BlockSpec((RB, 128), lambda i: (i, 0)),
      out_shape=jax.ShapeDtypeStruct((NPAD, 128), jnp.float32),
  )(aggp, degp, xr)


def _score_kernel(f, wp, pos_ref, neg_ref):
  s = f[0:BATCH]
  p = f[BATCH:2 * BATCH]
  n = f[2 * BATCH:3 * BATCH]
  w = wp[...]
  pos_ref[...] = jnp.sum(s * p * w, axis=1, keepdims=True)
  neg_ref[...] = jnp.sum(s * n * w, axis=1, keepdims=True)


def _score_call(feats, wp):
  return pl.pallas_call(
      _score_kernel,
      out_shape=[
          jax.ShapeDtypeStruct((BATCH, 1), jnp.float32),
          jax.ShapeDtypeStruct((BATCH, 1), jnp.float32),
      ],
  )(feats, wp)


def kernel(src_ids, pos_dst_ids, neg_dst_ids, node_feat, edge_index,
           Wn0, Wr0, b0, Wn1, Wr1, b1, w_pred):
  f32 = jnp.float32
  nf_pad = jnp.concatenate(
      [node_feat.astype(f32), jnp.zeros((NPAD - N, D_IN), f32)], axis=0)
  pad = N + (jnp.arange(EPAD - E, dtype=jnp.int32) % (NPAD - N))
  src = jnp.concatenate([edge_index[0].astype(jnp.int32), pad])
  dst = jnp.concatenate([edge_index[1].astype(jnp.int32), pad])
  dst3 = dst.reshape(NW, NCH, C)
  ids = jnp.concatenate([src_ids, pos_dst_ids, neg_dst_ids]).astype(jnp.int32)

  degpf = _deg_count(dst3)
  degp = degpf.reshape(NC, NPAD, 16)
  aggp0 = _edge_agg(nf_pad, src, dst).reshape(NC, NPAD, 128)

  y, xr = _layer_call(aggp0, degp, nf_pad, Wn0, Wr0,
                      b0.reshape(1, -1), Wn1, Wr1, b1.reshape(1, -1))

  aggp1 = _edge_agg(y, src, dst).reshape(NC, NPAD, 128)
  h1 = _h1_call(aggp1, degp, xr)

  feats = _gather_rows(h1, ids)
  pos, neg = _score_call(feats, w_pred.reshape(1, -1))
  return (pos.reshape(-1), neg.reshape(-1))


# speedup vs baseline: 2.5777x; 1.1840x over previous
import jax
import jax.numpy as jnp
from jax import lax
from jax.experimental import pallas as pl
from jax.experimental.pallas import tpu as pltpu
from jax.experimental.pallas import tpu_sc as plsc

N = 10000
NPAD = 10240
E = 320000
D_IN = 128
D_H = 256
D_OUT = 128
BATCH = 8192
IDS = 3 * BATCH

NC, NS = 2, 16
NW = NC * NS
C = 112
NCH = 90
EPW = NCH * C
EPAD = NW * EPW
RPT = NPAD // NS
RC = 80
IPW = IDS // NW
GC = 128

_mesh = plsc.VectorSubcoreMesh(
    core_axis_name="c", subcore_axis_name="s", num_cores=NC, num_subcores=NS)


def _make_edge_agg():
  scratch = [
      pltpu.VMEM((C,), jnp.int32),
      pltpu.VMEM((C,), jnp.int32),
      pltpu.VMEM((C,), jnp.int32),
      pltpu.VMEM((C,), jnp.int32),
      pltpu.VMEM((C, 128), jnp.float32),
      pltpu.VMEM((C, 128), jnp.float32),
      pltpu.VMEM((RC, 128), jnp.float32),
      pltpu.VMEM_SHARED((NPAD, 128), jnp.float32),
      pltpu.SemaphoreType.DMA,
      pltpu.SemaphoreType.DMA,
      pltpu.SemaphoreType.DMA,
      pltpu.SemaphoreType.DMA,
  ]

  def body(x_hbm, src_hbm, dst_hbm, agg_out, src_v0, src_v1, dst_v0, dst_v1,
           rows0, rows1, zbuf, acc_sh, sg0, sg1, ss0, ss1):
    src_v = (src_v0, src_v1)
    dst_v = (dst_v0, dst_v1)
    rows = (rows0, rows1)
    sg = (sg0, sg1)
    ss = (ss0, ss1)
    cid = lax.axis_index("c")
    sid = lax.axis_index("s")
    wid = sid * NC + cid
    z16 = jnp.zeros((16,), jnp.float32)

    def zrow(i, carry):
      for k in range(8):
        zbuf[i, pl.ds(k * 16, 16)] = z16
      return carry
    lax.fori_loop(0, RC, zrow, 0)
    row0 = sid * RPT
    for j in range(RPT // RC):
      pltpu.sync_copy(zbuf, acc_sh.at[pl.ds(row0 + j * RC, RC)])
    plsc.subcore_barrier()

    ebase = wid * EPW

    def ld_idx(i, b):
      pltpu.sync_copy(src_hbm.at[pl.ds(ebase + i * C, C)], src_v[b])
      pltpu.sync_copy(dst_hbm.at[pl.ds(ebase + i * C, C)], dst_v[b])

    def g_start(b):
      pltpu.async_copy(x_hbm.at[src_v[b]], rows[b], sg[b])

    def g_wait(b):
      pltpu.make_async_copy(x_hbm.at[src_v[b]], rows[b], sg[b]).wait()

    def s_start(b):
      pltpu.async_copy(rows[b], acc_sh.at[dst_v[b]], ss[b], add=True)

    def s_wait(b):
      pltpu.make_async_copy(rows[b], acc_sh.at[dst_v[b]], ss[b]).wait()

    T = NCH // 2
    ld_idx(0, 0)
    g_start(0)

    def super_chunk(t, carry):
      i0 = 2 * t

      @pl.when(t > 0)
      def _():
        s_wait(1)
      ld_idx(i0 + 1, 1)
      g_start(1)
      g_wait(0)
      s_start(0)
      g_wait(1)
      s_start(1)
      s_wait(0)

      @pl.when(t + 1 < T)
      def _():
        ld_idx(i0 + 2, 0)
        g_start(0)
      return carry
    lax.fori_loop(0, T, super_chunk, 0)
    s_wait(1)
    plsc.subcore_barrier()

    obase = cid * NPAD + row0
    for j in range(RPT // RC):
      pltpu.sync_copy(acc_sh.at[pl.ds(row0 + j * RC, RC)], zbuf)
      pltpu.sync_copy(zbuf, agg_out.at[pl.ds(obase + j * RC, RC)])

  return pl.kernel(
      body,
      out_type=jax.ShapeDtypeStruct((NC * NPAD, 128), jnp.float32),
      mesh=_mesh,
      scratch_types=scratch,
      compiler_params=pltpu.CompilerParams(use_tc_tiling_on_sc=False),
  )


def _deg_body(dst_hbm, deg_out, dst_all, ones_v, degst, deg_sh, sem):
  cid = lax.axis_index("c")
  sid = lax.axis_index("s")
  wid = sid * NC + cid
  z16 = jnp.zeros((16,), jnp.float32)
  row0 = sid * RPT

  pltpu.sync_copy(dst_hbm.at[wid], dst_all)

  def onesrow(i, carry):
    ones_v[i] = jnp.full((16,), 1.0, jnp.float32)
    return carry
  lax.fori_loop(0, C, onesrow, 0)

  def zdrow(i, carry):
    degst[i] = z16
    return carry
  lax.fori_loop(0, RPT, zdrow, 0)
  pltpu.sync_copy(degst, deg_sh.at[pl.ds(row0, RPT)])
  plsc.subcore_barrier()

  def chunk(i, carry):
    pltpu.async_copy(ones_v, deg_sh.at[dst_all.at[i]], sem, add=True)
    pltpu.make_async_copy(ones_v, deg_sh.at[dst_all.at[i]], sem).wait()
    return carry
  lax.fori_loop(0, NCH, chunk, 0)
  plsc.subcore_barrier()

  obase = cid * NPAD + row0
  pltpu.sync_copy(deg_sh.at[pl.ds(row0, RPT)], degst)
  pltpu.sync_copy(degst, deg_out.at[pl.ds(obase, RPT)])


_deg_count = pl.kernel(
    _deg_body,
    out_type=jax.ShapeDtypeStruct((NC * NPAD, 16), jnp.float32),
    mesh=_mesh,
    scratch_types=[
        pltpu.VMEM((NCH, C), jnp.int32),
        pltpu.VMEM((C, 16), jnp.float32),
        pltpu.VMEM((RPT, 16), jnp.float32),
        pltpu.VMEM_SHARED((NPAD, 16), jnp.float32),
        pltpu.SemaphoreType.DMA,
    ],
    compiler_params=pltpu.CompilerParams(use_tc_tiling_on_sc=False),
)

_edge_agg = _make_edge_agg()


def _gather_body(h_hbm, ids_hbm, out_hbm, idx_v, rows_v, sem):
  wid = lax.axis_index("s") * NC + lax.axis_index("c")
  base = wid * IPW
  for j in range(IPW // GC):
    pltpu.sync_copy(ids_hbm.at[pl.ds(base + j * GC, GC)], idx_v)
    pltpu.async_copy(h_hbm.at[idx_v], rows_v, sem).wait()
    pltpu.sync_copy(rows_v, out_hbm.at[pl.ds(base + j * GC, GC)])


_gather_rows = pl.kernel(
    _gather_body,
    out_type=jax.ShapeDtypeStruct((IDS, 128), jnp.float32),
    mesh=_mesh,
    scratch_types=[
        pltpu.VMEM((GC,), jnp.int32),
        pltpu.VMEM((GC, 128), jnp.float32),
        pltpu.SemaphoreType.DMA,
    ],
    compiler_params=pltpu.CompilerParams(use_tc_tiling_on_sc=False),
)


RB = 1280


def _layer_kernel(aggp, degp, nf, wn0, wr0, b0, wn1, wr1, b1, y_ref, xr_ref):
  agg = aggp[0] + aggp[1]
  deg = degp[0, :, 0:1] + degp[1, :, 0:1]
  rd = 1.0 / jnp.maximum(deg, 1.0)
  mean0 = agg * rd
  h = jnp.dot(mean0, wn0[...], preferred_element_type=jnp.float32)
  h = h + jnp.dot(nf[...], wr0[...], preferred_element_type=jnp.float32)
  h = jnp.maximum(h + b0[...], 0.0)
  y_ref[...] = jnp.dot(h, wn1[...], preferred_element_type=jnp.float32)
  xr_ref[...] = jnp.dot(h, wr1[...], preferred_element_type=jnp.float32) + b1[...]


def _layer_call(aggp, degp, nf, wn0, wr0, b0, wn1, wr1, b1):
  return pl.pallas_call(
      _layer_kernel,
      grid=(NPAD // RB,),
      in_specs=[
          pl.BlockSpec((NC, RB, 128), lambda i: (0, i, 0)),
          pl.BlockSpec((NC, RB, 16), lambda i: (0, i, 0)),
          pl.BlockSpec((RB, 128), lambda i: (i, 0)),
          pl.BlockSpec((D_IN, D_H), lambda i: (0, 0)),
          pl.BlockSpec((D_IN, D_H), lambda i: (0, 0)),
          pl.BlockSpec((1, D_H), lambda i: (0, 0)),
          pl.BlockSpec((D_H, D_OUT), lambda i: (0, 0)),
          pl.BlockSpec((D_H, D_OUT), lambda i: (0, 0)),
          pl.BlockSpec((1, D_OUT), lambda i: (0, 0)),
      ],
      out_specs=[
          pl.BlockSpec((RB, 128), lambda i: (i, 0)),
          pl.BlockSpec((RB, 128), lambda i: (i, 0)),
      ],
      out_shape=[
          jax.ShapeDtypeStruct((NPAD, 128), jnp.float32),
          jax.ShapeDtypeStruct((NPAD, 128), jnp.float32),
      ],
  )(aggp, degp, nf, wn0, wr0, b0, wn1, wr1, b1)


def _h1_kernel(aggp, degp, xr, out_ref):
  agg = aggp[0] + aggp[1]
  deg = degp[0, :, 0:1] + degp[1, :, 0:1]
  out_ref[...] = agg * (1.0 / jnp.maximum(deg, 1.0)) + xr[...]


def _h1_call(aggp, degp, xr):
  return pl.pallas_call(
      _h1_kernel,
      grid=(NPAD // RB,),
      in_specs=[
          pl.BlockSpec((NC, RB, 128), lambda i: (0, i, 0)),
          pl.BlockSpec((NC, RB, 16), lambda i: (0, i, 0)),
          pl.BlockSpec((RB, 128), lambda i: (i, 0)),
      ],
      out_specs=pl.BlockSpec((RB, 128), lambda i: (i, 0)),
      out_shape=jax.ShapeDtypeStruct((NPAD, 128), jnp.float32),
  )(aggp, degp, xr)


def _score_kernel(f, wp, pos_ref, neg_ref):
  s = f[0:BATCH]
  p = f[BATCH:2 * BATCH]
  n = f[2 * BATCH:3 * BATCH]
  w = wp[...]
  pos_ref[...] = jnp.sum(s * p * w, axis=1, keepdims=True)
  neg_ref[...] = jnp.sum(s * n * w, axis=1, keepdims=True)


def _score_call(feats, wp):
  return pl.pallas_call(
      _score_kernel,
      out_shape=[
          jax.ShapeDtypeStruct((BATCH, 1), jnp.float32),
          jax.ShapeDtypeStruct((BATCH, 1), jnp.float32),
      ],
  )(feats, wp)


def kernel(src_ids, pos_dst_ids, neg_dst_ids, node_feat, edge_index,
           Wn0, Wr0, b0, Wn1, Wr1, b1, w_pred):
  f32 = jnp.float32
  nf_pad = jnp.concatenate(
      [node_feat.astype(f32), jnp.zeros((NPAD - N, D_IN), f32)], axis=0)
  pad = N + (jnp.arange(EPAD - E, dtype=jnp.int32) % (NPAD - N))
  src = jnp.concatenate([edge_index[0].astype(jnp.int32), pad])
  dst = jnp.concatenate([edge_index[1].astype(jnp.int32), pad])
  dst3 = dst.reshape(NW, NCH, C)
  ids = jnp.concatenate([src_ids, pos_dst_ids, neg_dst_ids]).astype(jnp.int32)

  degpf = _deg_count(dst3)
  degp = degpf.reshape(NC, NPAD, 16)
  aggp0 = _edge_agg(nf_pad, src, dst).reshape(NC, NPAD, 128)

  y, xr = _layer_call(aggp0, degp, nf_pad, Wn0, Wr0,
                      b0.reshape(1, -1), Wn1, Wr1, b1.reshape(1, -1))

  aggp1 = _edge_agg(y, src, dst).reshape(NC, NPAD, 128)
  h1 = _h1_call(aggp1, degp, xr)

  feats = _gather_rows(h1, ids)
  pos, neg = _score_call(feats, w_pred.reshape(1, -1))
  return (pos.reshape(-1), neg.reshape(-1))


# speedup vs baseline: 2.6762x; 1.0382x over previous
import jax
import jax.numpy as jnp
from jax import lax
from jax.experimental import pallas as pl
from jax.experimental.pallas import tpu as pltpu
from jax.experimental.pallas import tpu_sc as plsc

N = 10000
NPAD = 10240
E = 320000
D_IN = 128
D_H = 256
D_OUT = 128
BATCH = 8192
IDS = 3 * BATCH

NC, NS = 2, 16
NW = NC * NS
C = 128
NCH = 80
EPW = NCH * C
EPAD = NW * EPW
RPT = NPAD // NS
RC = 80
IPW = IDS // NW
GC = 128

_mesh = plsc.VectorSubcoreMesh(
    core_axis_name="c", subcore_axis_name="s", num_cores=NC, num_subcores=NS)


def _make_edge_agg():
  scratch = [
      pltpu.VMEM((C,), jnp.int32),
      pltpu.VMEM((C,), jnp.int32),
      pltpu.VMEM((C,), jnp.int32),
      pltpu.VMEM((C,), jnp.int32),
      pltpu.VMEM((C, 128), jnp.float32),
      pltpu.VMEM((C, 128), jnp.float32),
      pltpu.VMEM((RC, 128), jnp.float32),
      pltpu.VMEM_SHARED((NPAD, 128), jnp.float32),
      pltpu.SemaphoreType.DMA,
      pltpu.SemaphoreType.DMA,
      pltpu.SemaphoreType.DMA,
      pltpu.SemaphoreType.DMA,
  ]

  def body(x_hbm, src_hbm, dst_hbm, agg_out, src_v0, src_v1, dst_v0, dst_v1,
           rows0, rows1, zbuf, acc_sh, sg0, sg1, ss0, ss1):
    src_v = (src_v0, src_v1)
    dst_v = (dst_v0, dst_v1)
    rows = (rows0, rows1)
    sg = (sg0, sg1)
    ss = (ss0, ss1)
    cid = lax.axis_index("c")
    sid = lax.axis_index("s")
    wid = sid * NC + cid
    z16 = jnp.zeros((16,), jnp.float32)

    def zrow(i, carry):
      for k in range(8):
        zbuf[i, pl.ds(k * 16, 16)] = z16
      return carry
    lax.fori_loop(0, RC, zrow, 0)
    row0 = sid * RPT
    for j in range(RPT // RC):
      pltpu.sync_copy(zbuf, acc_sh.at[pl.ds(row0 + j * RC, RC)])
    plsc.subcore_barrier()

    ebase = wid * EPW

    def ld_idx(i, b):
      pltpu.sync_copy(src_hbm.at[pl.ds(ebase + i * C, C)], src_v[b])
      pltpu.sync_copy(dst_hbm.at[pl.ds(ebase + i * C, C)], dst_v[b])

    def g_start(b):
      pltpu.async_copy(x_hbm.at[src_v[b]], rows[b], sg[b])

    def g_wait(b):
      pltpu.make_async_copy(x_hbm.at[src_v[b]], rows[b], sg[b]).wait()

    def s_start(b):
      pltpu.async_copy(rows[b], acc_sh.at[dst_v[b]], ss[b], add=True)

    def s_wait(b):
      pltpu.make_async_copy(rows[b], acc_sh.at[dst_v[b]], ss[b]).wait()

    T = NCH // 2
    ld_idx(0, 0)
    g_start(0)

    def super_chunk(t, carry):
      i0 = 2 * t

      @pl.when(t > 0)
      def _():
        s_wait(1)
      ld_idx(i0 + 1, 1)
      g_start(1)
      g_wait(0)
      s_start(0)
      g_wait(1)
      s_start(1)
      s_wait(0)

      @pl.when(t + 1 < T)
      def _():
        ld_idx(i0 + 2, 0)
        g_start(0)
      return carry
    lax.fori_loop(0, T, super_chunk, 0)
    s_wait(1)
    plsc.subcore_barrier()

    obase = cid * NPAD + row0
    for j in range(RPT // RC):
      pltpu.sync_copy(acc_sh.at[pl.ds(row0 + j * RC, RC)], zbuf)
      pltpu.sync_copy(zbuf, agg_out.at[pl.ds(obase + j * RC, RC)])

  return pl.kernel(
      body,
      out_type=jax.ShapeDtypeStruct((NC * NPAD, 128), jnp.float32),
      mesh=_mesh,
      scratch_types=scratch,
      compiler_params=pltpu.CompilerParams(use_tc_tiling_on_sc=False),
  )


def _deg_body(dst_hbm, deg_out, dst_all, ones_v, degst, deg_sh, sem):
  cid = lax.axis_index("c")
  sid = lax.axis_index("s")
  wid = sid * NC + cid
  z16 = jnp.zeros((16,), jnp.float32)
  row0 = sid * RPT

  pltpu.sync_copy(dst_hbm.at[wid], dst_all)

  def onesrow(i, carry):
    ones_v[i] = jnp.full((16,), 1.0, jnp.float32)
    return carry
  lax.fori_loop(0, C, onesrow, 0)

  def zdrow(i, carry):
    degst[i] = z16
    return carry
  lax.fori_loop(0, RPT, zdrow, 0)
  pltpu.sync_copy(degst, deg_sh.at[pl.ds(row0, RPT)])
  plsc.subcore_barrier()

  def chunk(i, carry):
    pltpu.async_copy(ones_v, deg_sh.at[dst_all.at[i]], sem, add=True)
    pltpu.make_async_copy(ones_v, deg_sh.at[dst_all.at[i]], sem).wait()
    return carry
  lax.fori_loop(0, NCH, chunk, 0)
  plsc.subcore_barrier()

  obase = cid * NPAD + row0
  pltpu.sync_copy(deg_sh.at[pl.ds(row0, RPT)], degst)
  pltpu.sync_copy(degst, deg_out.at[pl.ds(obase, RPT)])


_deg_count = pl.kernel(
    _deg_body,
    out_type=jax.ShapeDtypeStruct((NC * NPAD, 16), jnp.float32),
    mesh=_mesh,
    scratch_types=[
        pltpu.VMEM((NCH, C), jnp.int32),
        pltpu.VMEM((C, 16), jnp.float32),
        pltpu.VMEM((RPT, 16), jnp.float32),
        pltpu.VMEM_SHARED((NPAD, 16), jnp.float32),
        pltpu.SemaphoreType.DMA,
    ],
    compiler_params=pltpu.CompilerParams(use_tc_tiling_on_sc=False),
)

_edge_agg = _make_edge_agg()


def _gather_body(h_hbm, ids_hbm, out_hbm, idx_v, rows_v, sem):
  wid = lax.axis_index("s") * NC + lax.axis_index("c")
  base = wid * IPW
  for j in range(IPW // GC):
    pltpu.sync_copy(ids_hbm.at[pl.ds(base + j * GC, GC)], idx_v)
    pltpu.async_copy(h_hbm.at[idx_v], rows_v, sem).wait()
    pltpu.sync_copy(rows_v, out_hbm.at[pl.ds(base + j * GC, GC)])


_gather_rows = pl.kernel(
    _gather_body,
    out_type=jax.ShapeDtypeStruct((IDS, 128), jnp.float32),
    mesh=_mesh,
    scratch_types=[
        pltpu.VMEM((GC,), jnp.int32),
        pltpu.VMEM((GC, 128), jnp.float32),
        pltpu.SemaphoreType.DMA,
    ],
    compiler_params=pltpu.CompilerParams(use_tc_tiling_on_sc=False),
)


RB = 1280


def _layer_kernel(aggp, degp, nf, wn0, wr0, b0, wn1, wr1, b1, y_ref, xr_ref):
  agg = aggp[0] + aggp[1]
  deg = degp[0, :, 0:1] + degp[1, :, 0:1]
  rd = 1.0 / jnp.maximum(deg, 1.0)
  mean0 = agg * rd
  h = jnp.dot(mean0, wn0[...], preferred_element_type=jnp.float32)
  h = h + jnp.dot(nf[...], wr0[...], preferred_element_type=jnp.float32)
  h = jnp.maximum(h + b0[...], 0.0)
  y_ref[...] = jnp.dot(h, wn1[...], preferred_element_type=jnp.float32)
  xr_ref[...] = jnp.dot(h, wr1[...], preferred_element_type=jnp.float32) + b1[...]


def _layer_call(aggp, degp, nf, wn0, wr0, b0, wn1, wr1, b1):
  return pl.pallas_call(
      _layer_kernel,
      grid=(NPAD // RB,),
      in_specs=[
          pl.BlockSpec((NC, RB, 128), lambda i: (0, i, 0)),
          pl.BlockSpec((NC, RB, 16), lambda i: (0, i, 0)),
          pl.BlockSpec((RB, 128), lambda i: (i, 0)),
          pl.BlockSpec((D_IN, D_H), lambda i: (0, 0)),
          pl.BlockSpec((D_IN, D_H), lambda i: (0, 0)),
          pl.BlockSpec((1, D_H), lambda i: (0, 0)),
          pl.BlockSpec((D_H, D_OUT), lambda i: (0, 0)),
          pl.BlockSpec((D_H, D_OUT), lambda i: (0, 0)),
          pl.BlockSpec((1, D_OUT), lambda i: (0, 0)),
      ],
      out_specs=[
          pl.BlockSpec((RB, 128), lambda i: (i, 0)),
          pl.BlockSpec((RB, 128), lambda i: (i, 0)),
      ],
      out_shape=[
          jax.ShapeDtypeStruct((NPAD, 128), jnp.float32),
          jax.ShapeDtypeStruct((NPAD, 128), jnp.float32),
      ],
  )(aggp, degp, nf, wn0, wr0, b0, wn1, wr1, b1)


def _h1_kernel(aggp, degp, xr, out_ref):
  agg = aggp[0] + aggp[1]
  deg = degp[0, :, 0:1] + degp[1, :, 0:1]
  out_ref[...] = agg * (1.0 / jnp.maximum(deg, 1.0)) + xr[...]


def _h1_call(aggp, degp, xr):
  return pl.pallas_call(
      _h1_kernel,
      grid=(NPAD // RB,),
      in_specs=[
          pl.BlockSpec((NC, RB, 128), lambda i: (0, i, 0)),
          pl.BlockSpec((NC, RB, 16), lambda i: (0, i, 0)),
          pl.BlockSpec((RB, 128), lambda i: (i, 0)),
      ],
      out_specs=pl.BlockSpec((RB, 128), lambda i: (i, 0)),
      out_shape=jax.ShapeDtypeStruct((NPAD, 128), jnp.float32),
  )(aggp, degp, xr)


def _score_kernel(f, wp, pos_ref, neg_ref):
  s = f[0:BATCH]
  p = f[BATCH:2 * BATCH]
  n = f[2 * BATCH:3 * BATCH]
  w = wp[...]
  pos_ref[...] = jnp.sum(s * p * w, axis=1, keepdims=True)
  neg_ref[...] = jnp.sum(s * n * w, axis=1, keepdims=True)


def _score_call(feats, wp):
  return pl.pallas_call(
      _score_kernel,
      out_shape=[
          jax.ShapeDtypeStruct((BATCH, 1), jnp.float32),
          jax.ShapeDtypeStruct((BATCH, 1), jnp.float32),
      ],
  )(feats, wp)


def kernel(src_ids, pos_dst_ids, neg_dst_ids, node_feat, edge_index,
           Wn0, Wr0, b0, Wn1, Wr1, b1, w_pred):
  f32 = jnp.float32
  nf_pad = jnp.concatenate(
      [node_feat.astype(f32), jnp.zeros((NPAD - N, D_IN), f32)], axis=0)
  pad = N + (jnp.arange(EPAD - E, dtype=jnp.int32) % (NPAD - N))
  src = jnp.concatenate([edge_index[0].astype(jnp.int32), pad])
  dst = jnp.concatenate([edge_index[1].astype(jnp.int32), pad])
  dst3 = dst.reshape(NW, NCH, C)
  ids = jnp.concatenate([src_ids, pos_dst_ids, neg_dst_ids]).astype(jnp.int32)

  degpf = _deg_count(dst3)
  degp = degpf.reshape(NC, NPAD, 16)
  aggp0 = _edge_agg(nf_pad, src, dst).reshape(NC, NPAD, 128)

  y, xr = _layer_call(aggp0, degp, nf_pad, Wn0, Wr0,
                      b0.reshape(1, -1), Wn1, Wr1, b1.reshape(1, -1))

  aggp1 = _edge_agg(y, src, dst).reshape(NC, NPAD, 128)
  h1 = _h1_call(aggp1, degp, xr)

  feats = _gather_rows(h1, ids)
  pos, neg = _score_call(feats, w_pred.reshape(1, -1))
  return (pos.reshape(-1), neg.reshape(-1))


# speedup vs baseline: 2.7078x; 1.0118x over previous
import jax
import jax.numpy as jnp
from jax import lax
from jax.experimental import pallas as pl
from jax.experimental.pallas import tpu as pltpu
from jax.experimental.pallas import tpu_sc as plsc

N = 10000
NPAD = 10240
E = 320000
D_IN = 128
D_H = 256
D_OUT = 128
BATCH = 8192
IDS = 3 * BATCH

NC, NS = 2, 16
NW = NC * NS
C = 128
NCH = 80
EPW = NCH * C
EPAD = NW * EPW
RPT = NPAD // NS
RC = 80
IPW = IDS // NW
GC = 128

_mesh = plsc.VectorSubcoreMesh(
    core_axis_name="c", subcore_axis_name="s", num_cores=NC, num_subcores=NS)


def _make_edge_agg():
  scratch = [
      pltpu.VMEM((C,), jnp.int32),
      pltpu.VMEM((C,), jnp.int32),
      pltpu.VMEM((C,), jnp.int32),
      pltpu.VMEM((C,), jnp.int32),
      pltpu.VMEM((C, 128), jnp.float32),
      pltpu.VMEM((C, 128), jnp.float32),
      pltpu.VMEM_SHARED((NPAD, 128), jnp.float32),
      pltpu.SemaphoreType.DMA,
      pltpu.SemaphoreType.DMA,
      pltpu.SemaphoreType.DMA,
      pltpu.SemaphoreType.DMA,
  ]

  def body(x_hbm, src_hbm, dst_hbm, agg_out, src_v0, src_v1, dst_v0, dst_v1,
           rows0, rows1, acc_sh, sg0, sg1, ss0, ss1):
    src_v = (src_v0, src_v1)
    dst_v = (dst_v0, dst_v1)
    rows = (rows0, rows1)
    sg = (sg0, sg1)
    ss = (ss0, ss1)
    cid = lax.axis_index("c")
    sid = lax.axis_index("s")
    wid = sid * NC + cid
    z16 = jnp.zeros((16,), jnp.float32)

    def zrow(i, carry):
      for k in range(8):
        rows0[i, pl.ds(k * 16, 16)] = z16
      return carry
    lax.fori_loop(0, C, zrow, 0)
    row0 = sid * RPT
    for j in range(RPT // C):
      pltpu.async_copy(rows0, acc_sh.at[pl.ds(row0 + j * C, C)], ss0)
    for j in range(RPT // C):
      pltpu.make_async_copy(rows0, acc_sh.at[pl.ds(row0 + j * C, C)],
                            ss0).wait()
    plsc.subcore_barrier()

    ebase = wid * EPW

    def ld_idx(i, b):
      pltpu.sync_copy(src_hbm.at[pl.ds(ebase + i * C, C)], src_v[b])
      pltpu.sync_copy(dst_hbm.at[pl.ds(ebase + i * C, C)], dst_v[b])

    def g_start(b):
      pltpu.async_copy(x_hbm.at[src_v[b]], rows[b], sg[b])

    def g_wait(b):
      pltpu.make_async_copy(x_hbm.at[src_v[b]], rows[b], sg[b]).wait()

    def s_start(b):
      pltpu.async_copy(rows[b], acc_sh.at[dst_v[b]], ss[b], add=True)

    def s_wait(b):
      pltpu.make_async_copy(rows[b], acc_sh.at[dst_v[b]], ss[b]).wait()

    T = NCH // 2
    ld_idx(0, 0)
    g_start(0)

    def super_chunk(t, carry):
      i0 = 2 * t

      @pl.when(t > 0)
      def _():
        s_wait(1)
      ld_idx(i0 + 1, 1)
      g_start(1)
      g_wait(0)
      s_start(0)
      g_wait(1)
      s_start(1)
      s_wait(0)

      @pl.when(t + 1 < T)
      def _():
        ld_idx(i0 + 2, 0)
        g_start(0)
      return carry
    lax.fori_loop(0, T, super_chunk, 0)
    s_wait(1)
    plsc.subcore_barrier()

    obase = cid * NPAD + row0
    NR = RPT // C
    for j in range(NR):
      b = j % 2
      if j >= 2:
        pltpu.make_async_copy(
            rows[b], agg_out.at[pl.ds(obase + (j - 2) * C, C)], sg[b]).wait()
      pltpu.sync_copy(acc_sh.at[pl.ds(row0 + j * C, C)], rows[b])
      pltpu.async_copy(rows[b], agg_out.at[pl.ds(obase + j * C, C)], sg[b])
    for j in range(max(NR - 2, 0), NR):
      b = j % 2
      pltpu.make_async_copy(
          rows[b], agg_out.at[pl.ds(obase + j * C, C)], sg[b]).wait()

  return pl.kernel(
      body,
      out_type=jax.ShapeDtypeStruct((NC * NPAD, 128), jnp.float32),
      mesh=_mesh,
      scratch_types=scratch,
      compiler_params=pltpu.CompilerParams(use_tc_tiling_on_sc=False),
  )


def _deg_body(dst_hbm, deg_out, dst_all, ones_v, degst, deg_sh, sem):
  cid = lax.axis_index("c")
  sid = lax.axis_index("s")
  wid = sid * NC + cid
  z16 = jnp.zeros((16,), jnp.float32)
  row0 = sid * RPT

  pltpu.sync_copy(dst_hbm.at[wid], dst_all)

  def onesrow(i, carry):
    ones_v[i] = jnp.full((16,), 1.0, jnp.float32)
    return carry
  lax.fori_loop(0, C, onesrow, 0)

  def zdrow(i, carry):
    degst[i] = z16
    return carry
  lax.fori_loop(0, RPT, zdrow, 0)
  pltpu.sync_copy(degst, deg_sh.at[pl.ds(row0, RPT)])
  plsc.subcore_barrier()

  def chunk(i, carry):
    pltpu.async_copy(ones_v, deg_sh.at[dst_all.at[i]], sem, add=True)
    pltpu.make_async_copy(ones_v, deg_sh.at[dst_all.at[i]], sem).wait()
    return carry
  lax.fori_loop(0, NCH, chunk, 0)
  plsc.subcore_barrier()

  obase = cid * NPAD + row0
  pltpu.sync_copy(deg_sh.at[pl.ds(row0, RPT)], degst)
  pltpu.sync_copy(degst, deg_out.at[pl.ds(obase, RPT)])


_deg_count = pl.kernel(
    _deg_body,
    out_type=jax.ShapeDtypeStruct((NC * NPAD, 16), jnp.float32),
    mesh=_mesh,
    scratch_types=[
        pltpu.VMEM((NCH, C), jnp.int32),
        pltpu.VMEM((C, 16), jnp.float32),
        pltpu.VMEM((RPT, 16), jnp.float32),
        pltpu.VMEM_SHARED((NPAD, 16), jnp.float32),
        pltpu.SemaphoreType.DMA,
    ],
    compiler_params=pltpu.CompilerParams(use_tc_tiling_on_sc=False),
)

_edge_agg = _make_edge_agg()


def _gather_body(h_hbm, ids_hbm, out_hbm, idx_v, rows_v, sem):
  wid = lax.axis_index("s") * NC + lax.axis_index("c")
  base = wid * IPW
  for j in range(IPW // GC):
    pltpu.sync_copy(ids_hbm.at[pl.ds(base + j * GC, GC)], idx_v)
    pltpu.async_copy(h_hbm.at[idx_v], rows_v, sem).wait()
    pltpu.sync_copy(rows_v, out_hbm.at[pl.ds(base + j * GC, GC)])


_gather_rows = pl.kernel(
    _gather_body,
    out_type=jax.ShapeDtypeStruct((IDS, 128), jnp.float32),
    mesh=_mesh,
    scratch_types=[
        pltpu.VMEM((GC,), jnp.int32),
        pltpu.VMEM((GC, 128), jnp.float32),
        pltpu.SemaphoreType.DMA,
    ],
    compiler_params=pltpu.CompilerParams(use_tc_tiling_on_sc=False),
)


RB = 1280


def _layer_kernel(aggp, degp, nf, wn0, wr0, b0, wn1, wr1, b1, y_ref, xr_ref):
  agg = aggp[0] + aggp[1]
  deg = degp[0, :, 0:1] + degp[1, :, 0:1]
  rd = 1.0 / jnp.maximum(deg, 1.0)
  mean0 = agg * rd
  h = jnp.dot(mean0, wn0[...], preferred_element_type=jnp.float32)
  h = h + jnp.dot(nf[...], wr0[...], preferred_element_type=jnp.float32)
  h = jnp.maximum(h + b0[...], 0.0)
  y_ref[...] = jnp.dot(h, wn1[...], preferred_element_type=jnp.float32)
  xr_ref[...] = jnp.dot(h, wr1[...], preferred_element_type=jnp.float32) + b1[...]


def _layer_call(aggp, degp, nf, wn0, wr0, b0, wn1, wr1, b1):
  return pl.pallas_call(
      _layer_kernel,
      grid=(NPAD // RB,),
      in_specs=[
          pl.BlockSpec((NC, RB, 128), lambda i: (0, i, 0)),
          pl.BlockSpec((NC, RB, 16), lambda i: (0, i, 0)),
          pl.BlockSpec((RB, 128), lambda i: (i, 0)),
          pl.BlockSpec((D_IN, D_H), lambda i: (0, 0)),
          pl.BlockSpec((D_IN, D_H), lambda i: (0, 0)),
          pl.BlockSpec((1, D_H), lambda i: (0, 0)),
          pl.BlockSpec((D_H, D_OUT), lambda i: (0, 0)),
          pl.BlockSpec((D_H, D_OUT), lambda i: (0, 0)),
          pl.BlockSpec((1, D_OUT), lambda i: (0, 0)),
      ],
      out_specs=[
          pl.BlockSpec((RB, 128), lambda i: (i, 0)),
          pl.BlockSpec((RB, 128), lambda i: (i, 0)),
      ],
      out_shape=[
          jax.ShapeDtypeStruct((NPAD, 128), jnp.float32),
          jax.ShapeDtypeStruct((NPAD, 128), jnp.float32),
      ],
  )(aggp, degp, nf, wn0, wr0, b0, wn1, wr1, b1)


def _h1_kernel(aggp, degp, xr, out_ref):
  agg = aggp[0] + aggp[1]
  deg = degp[0, :, 0:1] + degp[1, :, 0:1]
  out_ref[...] = agg * (1.0 / jnp.maximum(deg, 1.0)) + xr[...]


def _h1_call(aggp, degp, xr):
  return pl.pallas_call(
      _h1_kernel,
      grid=(NPAD // RB,),
      in_specs=[
          pl.BlockSpec((NC, RB, 128), lambda i: (0, i, 0)),
          pl.BlockSpec((NC, RB, 16), lambda i: (0, i, 0)),
          pl.BlockSpec((RB, 128), lambda i: (i, 0)),
      ],
      out_specs=pl.BlockSpec((RB, 128), lambda i: (i, 0)),
      out_shape=jax.ShapeDtypeStruct((NPAD, 128), jnp.float32),
  )(aggp, degp, xr)


def _score_kernel(f, wp, pos_ref, neg_ref):
  s = f[0:BATCH]
  p = f[BATCH:2 * BATCH]
  n = f[2 * BATCH:3 * BATCH]
  w = wp[...]
  pos_ref[...] = jnp.sum(s * p * w, axis=1, keepdims=True)
  neg_ref[...] = jnp.sum(s * n * w, axis=1, keepdims=True)


def _score_call(feats, wp):
  return pl.pallas_call(
      _score_kernel,
      out_shape=[
          jax.ShapeDtypeStruct((BATCH, 1), jnp.float32),
          jax.ShapeDtypeStruct((BATCH, 1), jnp.float32),
      ],
  )(feats, wp)


def kernel(src_ids, pos_dst_ids, neg_dst_ids, node_feat, edge_index,
           Wn0, Wr0, b0, Wn1, Wr1, b1, w_pred):
  f32 = jnp.float32
  nf_pad = jnp.concatenate(
      [node_feat.astype(f32), jnp.zeros((NPAD - N, D_IN), f32)], axis=0)
  pad = N + (jnp.arange(EPAD - E, dtype=jnp.int32) % (NPAD - N))
  src = jnp.concatenate([edge_index[0].astype(jnp.int32), pad])
  dst = jnp.concatenate([edge_index[1].astype(jnp.int32), pad])
  dst3 = dst.reshape(NW, NCH, C)
  ids = jnp.concatenate([src_ids, pos_dst_ids, neg_dst_ids]).astype(jnp.int32)

  degpf = _deg_count(dst3)
  degp = degpf.reshape(NC, NPAD, 16)
  aggp0 = _edge_agg(nf_pad, src, dst).reshape(NC, NPAD, 128)

  y, xr = _layer_call(aggp0, degp, nf_pad, Wn0, Wr0,
                      b0.reshape(1, -1), Wn1, Wr1, b1.reshape(1, -1))

  aggp1 = _edge_agg(y, src, dst).reshape(NC, NPAD, 128)
  h1 = _h1_call(aggp1, degp, xr)

  feats = _gather_rows(h1, ids)
  pos, neg = _score_call(feats, w_pred.reshape(1, -1))
  return (pos.reshape(-1), neg.reshape(-1))


# speedup vs baseline: 2.7519x; 1.0163x over previous
import jax
import jax.numpy as jnp
from jax import lax
from jax.experimental import pallas as pl
from jax.experimental.pallas import tpu as pltpu
from jax.experimental.pallas import tpu_sc as plsc

N = 10000
NPAD = 10240
E = 320000
D_IN = 128
D_H = 256
D_OUT = 128
BATCH = 8192
IDS = 3 * BATCH

NC, NS = 2, 16
NW = NC * NS
C = 128
NCH = 80
EPW = NCH * C
EPAD = NW * EPW
RPT = NPAD // NS
RC = 80
IPW = IDS // NW
GC = 128

_mesh = plsc.VectorSubcoreMesh(
    core_axis_name="c", subcore_axis_name="s", num_cores=NC, num_subcores=NS)


def _make_edge_agg():
  scratch = [
      pltpu.VMEM((C,), jnp.int32),
      pltpu.VMEM((C,), jnp.int32),
      pltpu.VMEM((C,), jnp.int32),
      pltpu.VMEM((C,), jnp.int32),
      pltpu.VMEM((C, 128), jnp.float32),
      pltpu.VMEM((C, 128), jnp.float32),
      pltpu.VMEM_SHARED((NPAD, 128), jnp.float32),
      pltpu.SemaphoreType.DMA,
      pltpu.SemaphoreType.DMA,
      pltpu.SemaphoreType.DMA,
      pltpu.SemaphoreType.DMA,
  ]

  def body(x_hbm, src_hbm, dst_hbm, agg_out, src_v0, src_v1, dst_v0, dst_v1,
           rows0, rows1, acc_sh, sg0, sg1, ss0, ss1):
    src_v = (src_v0, src_v1)
    dst_v = (dst_v0, dst_v1)
    rows = (rows0, rows1)
    sg = (sg0, sg1)
    ss = (ss0, ss1)
    cid = lax.axis_index("c")
    sid = lax.axis_index("s")
    wid = sid * NC + cid
    z16 = jnp.zeros((16,), jnp.float32)

    def zrow(i, carry):
      for k in range(8):
        rows0[i, pl.ds(k * 16, 16)] = z16
      return carry
    lax.fori_loop(0, C, zrow, 0)
    row0 = sid * RPT
    for j in range(RPT // C):
      pltpu.async_copy(rows0, acc_sh.at[pl.ds(row0 + j * C, C)], ss0)
    for j in range(RPT // C):
      pltpu.make_async_copy(rows0, acc_sh.at[pl.ds(row0 + j * C, C)],
                            ss0).wait()
    plsc.subcore_barrier()

    ebase = wid * EPW

    def ld_idx(i, b):
      pltpu.sync_copy(src_hbm.at[pl.ds(ebase + i * C, C)], src_v[b])
      pltpu.sync_copy(dst_hbm.at[pl.ds(ebase + i * C, C)], dst_v[b])

    def g_start(b):
      pltpu.async_copy(x_hbm.at[src_v[b]], rows[b], sg[b])

    def g_wait(b):
      pltpu.make_async_copy(x_hbm.at[src_v[b]], rows[b], sg[b]).wait()

    def s_start(b):
      pltpu.async_copy(rows[b], acc_sh.at[dst_v[b]], ss[b], add=True)

    def s_wait(b):
      pltpu.make_async_copy(rows[b], acc_sh.at[dst_v[b]], ss[b]).wait()

    T = NCH // 2
    ld_idx(0, 0)
    g_start(0)

    def super_chunk(t, carry):
      i0 = 2 * t

      @pl.when(t > 0)
      def _():
        s_wait(1)
      ld_idx(i0 + 1, 1)
      g_start(1)
      g_wait(0)
      s_start(0)
      g_wait(1)
      s_start(1)
      s_wait(0)

      @pl.when(t + 1 < T)
      def _():
        ld_idx(i0 + 2, 0)
        g_start(0)
      return carry
    lax.fori_loop(0, T, super_chunk, 0)
    s_wait(1)
    plsc.subcore_barrier()

    obase = cid * NPAD + row0
    NR = RPT // C
    for j in range(NR):
      b = j % 2
      if j >= 2:
        pltpu.make_async_copy(
            rows[b], agg_out.at[pl.ds(obase + (j - 2) * C, C)], sg[b]).wait()
      pltpu.sync_copy(acc_sh.at[pl.ds(row0 + j * C, C)], rows[b])
      pltpu.async_copy(rows[b], agg_out.at[pl.ds(obase + j * C, C)], sg[b])
    for j in range(max(NR - 2, 0), NR):
      b = j % 2
      pltpu.make_async_copy(
          rows[b], agg_out.at[pl.ds(obase + j * C, C)], sg[b]).wait()

  return pl.kernel(
      body,
      out_type=jax.ShapeDtypeStruct((NC * NPAD, 128), jnp.float32),
      mesh=_mesh,
      scratch_types=scratch,
      compiler_params=pltpu.CompilerParams(use_tc_tiling_on_sc=False),
  )


def _deg_body(dst_hbm, deg_out, dst_all, ones_v, degst, deg_sh, sem):
  cid = lax.axis_index("c")
  sid = lax.axis_index("s")
  wid = sid * NC + cid
  z16 = jnp.zeros((16,), jnp.float32)
  row0 = sid * RPT

  pltpu.sync_copy(dst_hbm.at[wid], dst_all)

  def onesrow(i, carry):
    ones_v[i] = jnp.full((16,), 1.0, jnp.float32)
    return carry
  lax.fori_loop(0, C, onesrow, 0)

  def zdrow(i, carry):
    degst[i] = z16
    return carry
  lax.fori_loop(0, RPT, zdrow, 0)
  pltpu.sync_copy(degst, deg_sh.at[pl.ds(row0, RPT)])
  plsc.subcore_barrier()

  DGB = 8
  def chunk(t, carry):
    for k in range(DGB):
      pltpu.async_copy(ones_v, deg_sh.at[dst_all.at[t * DGB + k]], sem,
                       add=True)
    for k in range(DGB):
      pltpu.make_async_copy(ones_v, deg_sh.at[dst_all.at[t * DGB + k]],
                            sem).wait()
    return carry
  lax.fori_loop(0, NCH // DGB, chunk, 0)
  plsc.subcore_barrier()

  obase = cid * NPAD + row0
  pltpu.sync_copy(deg_sh.at[pl.ds(row0, RPT)], degst)
  pltpu.sync_copy(degst, deg_out.at[pl.ds(obase, RPT)])


_deg_count = pl.kernel(
    _deg_body,
    out_type=jax.ShapeDtypeStruct((NC * NPAD, 16), jnp.float32),
    mesh=_mesh,
    scratch_types=[
        pltpu.VMEM((NCH, C), jnp.int32),
        pltpu.VMEM((C, 16), jnp.float32),
        pltpu.VMEM((RPT, 16), jnp.float32),
        pltpu.VMEM_SHARED((NPAD, 16), jnp.float32),
        pltpu.SemaphoreType.DMA,
    ],
    compiler_params=pltpu.CompilerParams(use_tc_tiling_on_sc=False),
)

_edge_agg = _make_edge_agg()


def _gather_body(h_hbm, ids_hbm, out_hbm, idx_all, rowsa, rowsb,
                 sga, sgb, soa, sob):
  wid = lax.axis_index("s") * NC + lax.axis_index("c")
  base = wid * IPW
  rows = (rowsa, rowsb)
  sg = (sga, sgb)
  so = (soa, sob)
  NJ = IPW // GC
  pltpu.sync_copy(ids_hbm.at[pl.ds(base, IPW)], idx_all)

  def g_start(j, b):
    pltpu.async_copy(h_hbm.at[idx_all.at[pl.ds(j * GC, GC)]], rows[b], sg[b])

  def g_wait(j, b):
    pltpu.make_async_copy(h_hbm.at[idx_all.at[pl.ds(j * GC, GC)]], rows[b],
                          sg[b]).wait()

  def o_start(j, b):
    pltpu.async_copy(rows[b], out_hbm.at[pl.ds(base + j * GC, GC)], so[b])

  def o_wait(j, b):
    pltpu.make_async_copy(rows[b], out_hbm.at[pl.ds(base + j * GC, GC)],
                          so[b]).wait()

  g_start(0, 0)
  for j in range(NJ):
    b = j % 2
    g_wait(j, b)
    if j >= 2:
      o_wait(j - 2, b)
    o_start(j, b)
    if j + 1 < NJ:
      g_start(j + 1, b ^ 1)
  for j in range(max(NJ - 2, 0), NJ):
    o_wait(j, j % 2)


_gather_rows = pl.kernel(
    _gather_body,
    out_type=jax.ShapeDtypeStruct((IDS, 128), jnp.float32),
    mesh=_mesh,
    scratch_types=[
        pltpu.VMEM((IPW,), jnp.int32),
        pltpu.VMEM((GC, 128), jnp.float32),
        pltpu.VMEM((GC, 128), jnp.float32),
        pltpu.SemaphoreType.DMA,
        pltpu.SemaphoreType.DMA,
        pltpu.SemaphoreType.DMA,
        pltpu.SemaphoreType.DMA,
    ],
    compiler_params=pltpu.CompilerParams(use_tc_tiling_on_sc=False),
)


RB = 1280


def _layer_kernel(aggp, degp, nf, wn0, wr0, b0, wn1, wr1, b1, y_ref, xr_ref):
  agg = aggp[0] + aggp[1]
  deg = degp[0, :, 0:1] + degp[1, :, 0:1]
  rd = 1.0 / jnp.maximum(deg, 1.0)
  mean0 = agg * rd
  h = jnp.dot(mean0, wn0[...], preferred_element_type=jnp.float32)
  h = h + jnp.dot(nf[...], wr0[...], preferred_element_type=jnp.float32)
  h = jnp.maximum(h + b0[...], 0.0)
  y_ref[...] = jnp.dot(h, wn1[...], preferred_element_type=jnp.float32)
  xr_ref[...] = jnp.dot(h, wr1[...], preferred_element_type=jnp.float32) + b1[...]


def _layer_call(aggp, degp, nf, wn0, wr0, b0, wn1, wr1, b1):
  return pl.pallas_call(
      _layer_kernel,
      grid=(NPAD // RB,),
      in_specs=[
          pl.BlockSpec((NC, RB, 128), lambda i: (0, i, 0)),
          pl.BlockSpec((NC, RB, 16), lambda i: (0, i, 0)),
          pl.BlockSpec((RB, 128), lambda i: (i, 0)),
          pl.BlockSpec((D_IN, D_H), lambda i: (0, 0)),
          pl.BlockSpec((D_IN, D_H), lambda i: (0, 0)),
          pl.BlockSpec((1, D_H), lambda i: (0, 0)),
          pl.BlockSpec((D_H, D_OUT), lambda i: (0, 0)),
          pl.BlockSpec((D_H, D_OUT), lambda i: (0, 0)),
          pl.BlockSpec((1, D_OUT), lambda i: (0, 0)),
      ],
      out_specs=[
          pl.BlockSpec((RB, 128), lambda i: (i, 0)),
          pl.BlockSpec((RB, 128), lambda i: (i, 0)),
      ],
      out_shape=[
          jax.ShapeDtypeStruct((NPAD, 128), jnp.float32),
          jax.ShapeDtypeStruct((NPAD, 128), jnp.float32),
      ],
  )(aggp, degp, nf, wn0, wr0, b0, wn1, wr1, b1)


def _h1_kernel(aggp, degp, xr, out_ref):
  agg = aggp[0] + aggp[1]
  deg = degp[0, :, 0:1] + degp[1, :, 0:1]
  out_ref[...] = agg * (1.0 / jnp.maximum(deg, 1.0)) + xr[...]


def _h1_call(aggp, degp, xr):
  return pl.pallas_call(
      _h1_kernel,
      grid=(NPAD // RB,),
      in_specs=[
          pl.BlockSpec((NC, RB, 128), lambda i: (0, i, 0)),
          pl.BlockSpec((NC, RB, 16), lambda i: (0, i, 0)),
          pl.BlockSpec((RB, 128), lambda i: (i, 0)),
      ],
      out_specs=pl.BlockSpec((RB, 128), lambda i: (i, 0)),
      out_shape=jax.ShapeDtypeStruct((NPAD, 128), jnp.float32),
  )(aggp, degp, xr)


def _score_kernel(f, wp, pos_ref, neg_ref):
  s = f[0:BATCH]
  p = f[BATCH:2 * BATCH]
  n = f[2 * BATCH:3 * BATCH]
  w = wp[...]
  pos_ref[...] = jnp.sum(s * p * w, axis=1, keepdims=True)
  neg_ref[...] = jnp.sum(s * n * w, axis=1, keepdims=True)


def _score_call(feats, wp):
  return pl.pallas_call(
      _score_kernel,
      out_shape=[
          jax.ShapeDtypeStruct((BATCH, 1), jnp.float32),
          jax.ShapeDtypeStruct((BATCH, 1), jnp.float32),
      ],
  )(feats, wp)


def kernel(src_ids, pos_dst_ids, neg_dst_ids, node_feat, edge_index,
           Wn0, Wr0, b0, Wn1, Wr1, b1, w_pred):
  f32 = jnp.float32
  nf_pad = jnp.concatenate(
      [node_feat.astype(f32), jnp.zeros((NPAD - N, D_IN), f32)], axis=0)
  pad = N + (jnp.arange(EPAD - E, dtype=jnp.int32) % (NPAD - N))
  src = jnp.concatenate([edge_index[0].astype(jnp.int32), pad])
  dst = jnp.concatenate([edge_index[1].astype(jnp.int32), pad])
  dst3 = dst.reshape(NW, NCH, C)
  ids = jnp.concatenate([src_ids, pos_dst_ids, neg_dst_ids]).astype(jnp.int32)

  degpf = _deg_count(dst3)
  degp = degpf.reshape(NC, NPAD, 16)
  aggp0 = _edge_agg(nf_pad, src, dst).reshape(NC, NPAD, 128)

  y, xr = _layer_call(aggp0, degp, nf_pad, Wn0, Wr0,
                      b0.reshape(1, -1), Wn1, Wr1, b1.reshape(1, -1))

  aggp1 = _edge_agg(y, src, dst).reshape(NC, NPAD, 128)
  h1 = _h1_call(aggp1, degp, xr)

  feats = _gather_rows(h1, ids)
  pos, neg = _score_call(feats, w_pred.reshape(1, -1))
  return (pos.reshape(-1), neg.reshape(-1))


# speedup vs baseline: 3.0040x; 1.0916x over previous
import jax
import jax.numpy as jnp
from jax import lax
from jax.experimental import pallas as pl
from jax.experimental.pallas import tpu as pltpu
from jax.experimental.pallas import tpu_sc as plsc

N = 10000
NPAD = 10240
E = 320000
D_IN = 128
D_H = 256
D_OUT = 128
BATCH = 8192
IDS = 3 * BATCH

NC, NS = 2, 16
NW = NC * NS
C = 128
NCH = 80
EPW = NCH * C
EPAD = NW * EPW
RPT = NPAD // NS
RC = 80
IPW = IDS // NW
GC = 128

_mesh = plsc.VectorSubcoreMesh(
    core_axis_name="c", subcore_axis_name="s", num_cores=NC, num_subcores=NS)


def _make_edge_agg():
  scratch = [
      pltpu.VMEM((C,), jnp.int32),
      pltpu.VMEM((C,), jnp.int32),
      pltpu.VMEM((C,), jnp.int32),
      pltpu.VMEM((C,), jnp.int32),
      pltpu.VMEM((C, 128), jnp.float32),
      pltpu.VMEM((C, 128), jnp.float32),
      pltpu.VMEM_SHARED((NPAD, 128), jnp.float32),
      pltpu.SemaphoreType.DMA,
      pltpu.SemaphoreType.DMA,
      pltpu.SemaphoreType.DMA,
      pltpu.SemaphoreType.DMA,
  ]

  def body(x_hbm, src_hbm, dst_hbm, agg_out, src_v0, src_v1, dst_v0, dst_v1,
           rows0, rows1, acc_sh, sg0, sg1, ss0, ss1):
    src_v = (src_v0, src_v1)
    dst_v = (dst_v0, dst_v1)
    rows = (rows0, rows1)
    sg = (sg0, sg1)
    ss = (ss0, ss1)
    cid = lax.axis_index("c")
    sid = lax.axis_index("s")
    wid = sid * NC + cid
    z16 = jnp.zeros((16,), jnp.float32)

    def zrow(i, carry):
      for k in range(8):
        rows0[i, pl.ds(k * 16, 16)] = z16
      return carry
    lax.fori_loop(0, C, zrow, 0)
    row0 = sid * RPT
    for j in range(RPT // C):
      pltpu.async_copy(rows0, acc_sh.at[pl.ds(row0 + j * C, C)], ss0)
    for j in range(RPT // C):
      pltpu.make_async_copy(rows0, acc_sh.at[pl.ds(row0 + j * C, C)],
                            ss0).wait()
    plsc.subcore_barrier()

    ebase = wid * EPW

    def ld_src(i, b):
      pltpu.sync_copy(src_hbm.at[pl.ds(ebase + i * C, C)], src_v[b])

    def ld_dst(i, b):
      pltpu.sync_copy(dst_hbm.at[pl.ds(ebase + i * C, C)], dst_v[b])

    def g_start(b):
      pltpu.async_copy(x_hbm.at[src_v[b]], rows[b], sg[b])

    def g_wait(b):
      pltpu.make_async_copy(x_hbm.at[src_v[b]], rows[b], sg[b]).wait()

    def s_start(b):
      pltpu.async_copy(rows[b], acc_sh.at[dst_v[b]], ss[b], add=True)

    def s_wait(b):
      pltpu.make_async_copy(rows[b], acc_sh.at[dst_v[b]], ss[b]).wait()

    T = NCH // 2
    ld_src(0, 0)
    g_start(0)

    def super_chunk(t, carry):
      i0 = 2 * t
      ld_dst(i0, 0)
      ld_src(i0 + 1, 1)

      @pl.when(t > 0)
      def _():
        s_wait(1)
      g_start(1)
      ld_dst(i0 + 1, 1)
      g_wait(0)
      s_start(0)
      g_wait(1)
      s_start(1)
      s_wait(0)

      @pl.when(t + 1 < T)
      def _():
        ld_src(i0 + 2, 0)
        g_start(0)
      return carry
    lax.fori_loop(0, T, super_chunk, 0)
    s_wait(1)
    plsc.subcore_barrier()

    obase = cid * NPAD + row0
    NR = RPT // C
    for j in range(NR):
      b = j % 2
      if j >= 2:
        pltpu.make_async_copy(
            rows[b], agg_out.at[pl.ds(obase + (j - 2) * C, C)], sg[b]).wait()
      pltpu.sync_copy(acc_sh.at[pl.ds(row0 + j * C, C)], rows[b])
      pltpu.async_copy(rows[b], agg_out.at[pl.ds(obase + j * C, C)], sg[b])
    for j in range(max(NR - 2, 0), NR):
      b = j % 2
      pltpu.make_async_copy(
          rows[b], agg_out.at[pl.ds(obase + j * C, C)], sg[b]).wait()

  return pl.kernel(
      body,
      out_type=jax.ShapeDtypeStruct((NC * NPAD, 128), jnp.float32),
      mesh=_mesh,
      scratch_types=scratch,
      compiler_params=pltpu.CompilerParams(use_tc_tiling_on_sc=False),
  )


def _deg_body(dst_hbm, deg_out, dst_all, ones_v, degst, deg_sh, sem):
  cid = lax.axis_index("c")
  sid = lax.axis_index("s")
  wid = sid * NC + cid
  z16 = jnp.zeros((16,), jnp.float32)
  row0 = sid * RPT

  pltpu.sync_copy(dst_hbm.at[wid], dst_all)

  def onesrow(i, carry):
    ones_v[i] = jnp.full((16,), 1.0, jnp.float32)
    return carry
  lax.fori_loop(0, C, onesrow, 0)

  def zdrow(i, carry):
    degst[i] = z16
    return carry
  lax.fori_loop(0, RPT, zdrow, 0)
  pltpu.sync_copy(degst, deg_sh.at[pl.ds(row0, RPT)])
  plsc.subcore_barrier()

  DGB = 8
  def chunk(t, carry):
    for k in range(DGB):
      pltpu.async_copy(ones_v, deg_sh.at[dst_all.at[t * DGB + k]], sem,
                       add=True)
    for k in range(DGB):
      pltpu.make_async_copy(ones_v, deg_sh.at[dst_all.at[t * DGB + k]],
                            sem).wait()
    return carry
  lax.fori_loop(0, NCH // DGB, chunk, 0)
  plsc.subcore_barrier()

  obase = cid * NPAD + row0
  pltpu.sync_copy(deg_sh.at[pl.ds(row0, RPT)], degst)
  pltpu.sync_copy(degst, deg_out.at[pl.ds(obase, RPT)])


_deg_count = pl.kernel(
    _deg_body,
    out_type=jax.ShapeDtypeStruct((NC * NPAD, 16), jnp.float32),
    mesh=_mesh,
    scratch_types=[
        pltpu.VMEM((NCH, C), jnp.int32),
        pltpu.VMEM((C, 16), jnp.float32),
        pltpu.VMEM((RPT, 16), jnp.float32),
        pltpu.VMEM_SHARED((NPAD, 16), jnp.float32),
        pltpu.SemaphoreType.DMA,
    ],
    compiler_params=pltpu.CompilerParams(use_tc_tiling_on_sc=False),
)

_edge_agg = _make_edge_agg()


def _gather_body(h_hbm, ids_hbm, out_hbm, idx_all, rowsa, rowsb,
                 sga, sgb, soa, sob):
  wid = lax.axis_index("s") * NC + lax.axis_index("c")
  base = wid * IPW
  rows = (rowsa, rowsb)
  sg = (sga, sgb)
  so = (soa, sob)
  NJ = IPW // GC
  pltpu.sync_copy(ids_hbm.at[pl.ds(base, IPW)], idx_all)

  def g_start(j, b):
    pltpu.async_copy(h_hbm.at[idx_all.at[pl.ds(j * GC, GC)]], rows[b], sg[b])

  def g_wait(j, b):
    pltpu.make_async_copy(h_hbm.at[idx_all.at[pl.ds(j * GC, GC)]], rows[b],
                          sg[b]).wait()

  def o_start(j, b):
    pltpu.async_copy(rows[b], out_hbm.at[pl.ds(base + j * GC, GC)], so[b])

  def o_wait(j, b):
    pltpu.make_async_copy(rows[b], out_hbm.at[pl.ds(base + j * GC, GC)],
                          so[b]).wait()

  g_start(0, 0)
  for j in range(NJ):
    b = j % 2
    g_wait(j, b)
    if j >= 2:
      o_wait(j - 2, b)
    o_start(j, b)
    if j + 1 < NJ:
      g_start(j + 1, b ^ 1)
  for j in range(max(NJ - 2, 0), NJ):
    o_wait(j, j % 2)


_gather_rows = pl.kernel(
    _gather_body,
    out_type=jax.ShapeDtypeStruct((IDS, 128), jnp.float32),
    mesh=_mesh,
    scratch_types=[
        pltpu.VMEM((IPW,), jnp.int32),
        pltpu.VMEM((GC, 128), jnp.float32),
        pltpu.VMEM((GC, 128), jnp.float32),
        pltpu.SemaphoreType.DMA,
        pltpu.SemaphoreType.DMA,
        pltpu.SemaphoreType.DMA,
        pltpu.SemaphoreType.DMA,
    ],
    compiler_params=pltpu.CompilerParams(use_tc_tiling_on_sc=False),
)


RB = 1280


def _layer_kernel(aggp, degp, nf, wn0, wr0, b0, wn1, wr1, b1, y_ref, xr_ref):
  agg = aggp[0] + aggp[1]
  deg = degp[0, :, 0:1] + degp[1, :, 0:1]
  rd = 1.0 / jnp.maximum(deg, 1.0)
  mean0 = agg * rd
  h = jnp.dot(mean0, wn0[...], preferred_element_type=jnp.float32)
  h = h + jnp.dot(nf[...], wr0[...], preferred_element_type=jnp.float32)
  h = jnp.maximum(h + b0[...], 0.0)
  y_ref[...] = jnp.dot(h, wn1[...], preferred_element_type=jnp.float32)
  xr_ref[...] = jnp.dot(h, wr1[...], preferred_element_type=jnp.float32) + b1[...]


def _layer_call(aggp, degp, nf, wn0, wr0, b0, wn1, wr1, b1):
  return pl.pallas_call(
      _layer_kernel,
      grid=(NPAD // RB,),
      in_specs=[
          pl.BlockSpec((NC, RB, 128), lambda i: (0, i, 0)),
          pl.BlockSpec((NC, RB, 16), lambda i: (0, i, 0)),
          pl.BlockSpec((RB, 128), lambda i: (i, 0)),
          pl.BlockSpec((D_IN, D_H), lambda i: (0, 0)),
          pl.BlockSpec((D_IN, D_H), lambda i: (0, 0)),
          pl.BlockSpec((1, D_H), lambda i: (0, 0)),
          pl.BlockSpec((D_H, D_OUT), lambda i: (0, 0)),
          pl.BlockSpec((D_H, D_OUT), lambda i: (0, 0)),
          pl.BlockSpec((1, D_OUT), lambda i: (0, 0)),
      ],
      out_specs=[
          pl.BlockSpec((RB, 128), lambda i: (i, 0)),
          pl.BlockSpec((RB, 128), lambda i: (i, 0)),
      ],
      out_shape=[
          jax.ShapeDtypeStruct((NPAD, 128), jnp.float32),
          jax.ShapeDtypeStruct((NPAD, 128), jnp.float32),
      ],
  )(aggp, degp, nf, wn0, wr0, b0, wn1, wr1, b1)


def _h1_kernel(aggp, degp, xr, out_ref):
  agg = aggp[0] + aggp[1]
  deg = degp[0, :, 0:1] + degp[1, :, 0:1]
  out_ref[...] = agg * (1.0 / jnp.maximum(deg, 1.0)) + xr[...]


def _h1_call(aggp, degp, xr):
  return pl.pallas_call(
      _h1_kernel,
      grid=(NPAD // RB,),
      in_specs=[
          pl.BlockSpec((NC, RB, 128), lambda i: (0, i, 0)),
          pl.BlockSpec((NC, RB, 16), lambda i: (0, i, 0)),
          pl.BlockSpec((RB, 128), lambda i: (i, 0)),
      ],
      out_specs=pl.BlockSpec((RB, 128), lambda i: (i, 0)),
      out_shape=jax.ShapeDtypeStruct((NPAD, 128), jnp.float32),
  )(aggp, degp, xr)


def _score_kernel(f, wp, pos_ref, neg_ref):
  s = f[0:BATCH]
  p = f[BATCH:2 * BATCH]
  n = f[2 * BATCH:3 * BATCH]
  w = wp[...]
  pos_ref[...] = jnp.sum(s * p * w, axis=1, keepdims=True)
  neg_ref[...] = jnp.sum(s * n * w, axis=1, keepdims=True)


def _score_call(feats, wp):
  return pl.pallas_call(
      _score_kernel,
      out_shape=[
          jax.ShapeDtypeStruct((BATCH, 1), jnp.float32),
          jax.ShapeDtypeStruct((BATCH, 1), jnp.float32),
      ],
  )(feats, wp)


def kernel(src_ids, pos_dst_ids, neg_dst_ids, node_feat, edge_index,
           Wn0, Wr0, b0, Wn1, Wr1, b1, w_pred):
  f32 = jnp.float32
  nf_pad = jnp.concatenate(
      [node_feat.astype(f32), jnp.zeros((NPAD - N, D_IN), f32)], axis=0)
  pad = N + (jnp.arange(EPAD - E, dtype=jnp.int32) % (NPAD - N))
  src = jnp.concatenate([edge_index[0].astype(jnp.int32), pad])
  dst = jnp.concatenate([edge_index[1].astype(jnp.int32), pad])
  dst3 = dst.reshape(NW, NCH, C)
  ids = jnp.concatenate([src_ids, pos_dst_ids, neg_dst_ids]).astype(jnp.int32)

  degpf = _deg_count(dst3)
  degp = degpf.reshape(NC, NPAD, 16)
  aggp0 = _edge_agg(nf_pad, src, dst).reshape(NC, NPAD, 128)

  y, xr = _layer_call(aggp0, degp, nf_pad, Wn0, Wr0,
                      b0.reshape(1, -1), Wn1, Wr1, b1.reshape(1, -1))

  aggp1 = _edge_agg(y, src, dst).reshape(NC, NPAD, 128)
  h1 = _h1_call(aggp1, degp, xr)

  feats = _gather_rows(h1, ids)
  pos, neg = _score_call(feats, w_pred.reshape(1, -1))
  return (pos.reshape(-1), neg.reshape(-1))
